# baseline (speedup 1.0000x reference)
import functools
import math

import numpy as np
import jax
import jax.numpy as jnp
from jax import lax
from jax.experimental import pallas as pl
from jax.experimental.pallas import tpu as pltpu
from jax.experimental.pallas import tpu_sc as plsc

F32 = jnp.float32

_NC, _NS, _NL = 2, 16, 16
_NW = _NC * _NS


def _posenc(seq, dim):
    pos = np.arange(seq)[:, None].astype(np.float32)
    div = np.exp(np.arange(0, dim, 2).astype(np.float32) * (-math.log(10000.0) / dim))
    pe = np.zeros((seq, dim), np.float32)
    pe[:, 0::2] = np.sin(pos * div)
    pe[:, 1::2] = np.cos(pos * div)
    return jnp.asarray(pe)


def _scalepe_body(x_ref, p_ref, o_ref, *, scale):
    o_ref[...] = x_ref[...] * scale + p_ref[...]


def _scale_pe(x, pe, scale, mb=256):
    M, D = x.shape
    return pl.pallas_call(
        functools.partial(_scalepe_body, scale=scale),
        grid=(M // mb,),
        in_specs=[
            pl.BlockSpec((mb, D), lambda i: (i, 0)),
            pl.BlockSpec((mb, D), lambda i: (i, 0)),
        ],
        out_specs=pl.BlockSpec((mb, D), lambda i: (i, 0)),
        out_shape=jax.ShapeDtypeStruct((M, D), F32),
    )(x, pe)


def _mm_body(x_ref, w_ref, b_ref, o_ref, *, act, bf16):
    if bf16:
        acc = jnp.dot(x_ref[...].astype(jnp.bfloat16),
                      w_ref[...].astype(jnp.bfloat16),
                      preferred_element_type=F32)
    else:
        acc = jnp.dot(x_ref[...], w_ref[...], preferred_element_type=F32)
    acc = acc + b_ref[...]
    if act == "relu":
        acc = jnp.maximum(acc, 0.0)
    elif act == "sigmoid":
        acc = 1.0 / (1.0 + jnp.exp(-acc))
    o_ref[...] = acc


def _matmul(x, w, b, act="none", mb=256, bf16=True):
    M, K = x.shape
    _, N = w.shape
    mb = min(mb, M)
    return pl.pallas_call(
        functools.partial(_mm_body, act=act, bf16=bf16),
        grid=(M // mb,),
        in_specs=[
            pl.BlockSpec((mb, K), lambda i: (i, 0)),
            pl.BlockSpec((K, N), lambda i: (0, 0)),
            pl.BlockSpec((1, N), lambda i: (0, 0)),
        ],
        out_specs=pl.BlockSpec((mb, N), lambda i: (i, 0)),
        out_shape=jax.ShapeDtypeStruct((M, N), F32),
    )(x, w, b.reshape(1, N))


def _attn_body(q_ref, k_ref, v_ref, o_ref, *, sb, dh, S):
    i = pl.program_id(1)
    q = q_ref[...].astype(jnp.bfloat16)
    k = k_ref[...].astype(jnp.bfloat16)
    v = v_ref[...].astype(jnp.bfloat16)
    s = lax.dot_general(q, k, (((1,), (1,)), ((), ())), preferred_element_type=F32)
    s = s * (1.0 / math.sqrt(dh))
    rows = lax.broadcasted_iota(jnp.int32, (sb, S), 0) + i * sb
    cols = lax.broadcasted_iota(jnp.int32, (sb, S), 1)
    s = jnp.where(cols > rows, -1e30, s)
    m = jnp.max(s, axis=1, keepdims=True)
    p = jnp.exp(s - m)
    p = p / jnp.sum(p, axis=1, keepdims=True)
    o_ref[...] = jnp.dot(p.astype(jnp.bfloat16), v, preferred_element_type=F32)


def _attention(qkv, S, D, nhead, sb=256):
    dh = D // nhead
    return pl.pallas_call(
        functools.partial(_attn_body, sb=sb, dh=dh, S=S),
        grid=(nhead, S // sb),
        in_specs=[
            pl.BlockSpec((sb, dh), lambda h, i: (i, h)),
            pl.BlockSpec((S, dh), lambda h, i: (0, nhead + h)),
            pl.BlockSpec((S, dh), lambda h, i: (0, 2 * nhead + h)),
        ],
        out_specs=pl.BlockSpec((sb, dh), lambda h, i: (i, h)),
        out_shape=jax.ShapeDtypeStruct((S, D), F32),
    )(qkv, qkv, qkv)


def _addln_body(a_ref, b_ref, w_ref, bb_ref, o_ref):
    x = a_ref[...] + b_ref[...]
    m = jnp.mean(x, axis=1, keepdims=True)
    var = jnp.mean((x - m) ** 2, axis=1, keepdims=True)
    o_ref[...] = (x - m) / jnp.sqrt(var + 1e-5) * w_ref[...] + bb_ref[...]


def _add_ln(a, b, w, bias, mb=256):
    M, D = a.shape
    return pl.pallas_call(
        _addln_body,
        grid=(M // mb,),
        in_specs=[
            pl.BlockSpec((mb, D), lambda i: (i, 0)),
            pl.BlockSpec((mb, D), lambda i: (i, 0)),
            pl.BlockSpec((1, D), lambda i: (0, 0)),
            pl.BlockSpec((1, D), lambda i: (0, 0)),
        ],
        out_specs=pl.BlockSpec((mb, D), lambda i: (i, 0)),
        out_shape=jax.ShapeDtypeStruct((M, D), F32),
    )(a, b, w.reshape(1, D), bias.reshape(1, D))


def _topk_body(s_ref, o_ref, *, nq):
    s = s_ref[...]
    R, C = s.shape
    flat = (lax.broadcasted_iota(jnp.int32, (R, C), 0) * C
            + lax.broadcasted_iota(jnp.int32, (R, C), 1))
    rowi = lax.broadcasted_iota(jnp.int32, (64, 128), 0)
    coli = lax.broadcasted_iota(jnp.int32, (64, 128), 1)

    def body(t, carry):
        s, o = carry
        m = jnp.max(s)
        cand = jnp.where(s == m, flat, jnp.int32(2 ** 30))
        amin = jnp.min(cand)
        o = o + jnp.where(rowi == t, m, 0.0)
        s = jnp.where(flat == amin, jnp.float32(-1e30), s)
        return s, o

    s, o = lax.fori_loop(0, nq, body, (s, jnp.zeros((64, 128), F32)))
    total = jnp.sum(jnp.where(coli == 0, o, 0.0))
    o_ref[...] = o / total


def _topk(score2d, nq):
    return pl.pallas_call(
        functools.partial(_topk_body, nq=nq),
        out_shape=jax.ShapeDtypeStruct((64, 128), F32),
    )(score2d)


def _contrib_body(w_ref, rv_ref, ri_ref, oc_ref, oi_ref, *, sb, npair, nrow):
    rv = rv_ref[...]
    ri = ri_ref[...]
    cps = []
    ips = []
    for j in range(npair):
        a = jnp.log(w_ref[2 * j, 0, 0] * rv[2 * j, 0] + 1e-40)
        b = jnp.log(w_ref[2 * j + 1, 0, 0] * rv[2 * j + 1, 0] + 1e-40)
        cps.append(jnp.concatenate([a, b], axis=1)[:, None, :])
        ips.append(jnp.concatenate([ri[2 * j, 0], ri[2 * j + 1, 0]],
                                   axis=1)[:, None, :])
    pad = nrow - npair
    cps.append(jnp.zeros((sb, pad, 128), F32))
    ips.append(jnp.zeros((sb, pad, 128), jnp.int32))
    oc_ref[...] = jnp.concatenate(cps, axis=1)
    oi_ref[...] = jnp.concatenate(ips, axis=1)


def _contrib(w2d, rv4, ri4, sb=128, nrow=32):
    NQ_, _, S_, TK = rv4.shape
    npair = NQ_ // 2
    w3 = w2d.reshape(64, 1, 128)
    return pl.pallas_call(
        functools.partial(_contrib_body, sb=sb, npair=npair, nrow=nrow),
        grid=(S_ // sb,),
        in_specs=[
            pl.BlockSpec((64, 1, 128), lambda s: (0, 0, 0)),
            pl.BlockSpec((NQ_, 1, sb, TK), lambda s: (0, 0, s, 0)),
            pl.BlockSpec((NQ_, 1, sb, TK), lambda s: (0, 0, s, 0)),
        ],
        out_specs=(pl.BlockSpec((sb, nrow, 128), lambda s: (s, 0, 0)),
                   pl.BlockSpec((sb, nrow, 128), lambda s: (s, 0, 0))),
        out_shape=(jax.ShapeDtypeStruct((S_, nrow, 128), F32),
                   jax.ShapeDtypeStruct((S_, nrow, 128), jnp.int32)),
    )(w3, rv4, ri4)


def _loss_body(p_ref, a_ref, o_ref, *, V):
    part = jnp.sum(p_ref[...], axis=1, keepdims=True)
    alab = a_ref[...][:, 0:1]
    lr = jnp.log(V * math.e + part) - 1.0 - alab
    o_ref[...] = jnp.mean(lr).reshape(1, 1)


def _loss(part, alab, V):
    return pl.pallas_call(
        functools.partial(_loss_body, V=V),
        out_shape=jax.ShapeDtypeStruct((1, 1), F32),
    )(part, alab)


def _sc_mesh():
    return plsc.VectorSubcoreMesh(
        core_axis_name="c", subcore_axis_name="s",
        num_cores=_NC, num_subcores=_NS)


def _sc_embed_gather(idx, emb):
    (Sn,) = idx.shape
    V, D = emb.shape
    bpw = Sn // _NW

    @functools.partial(
        pl.kernel, mesh=_sc_mesh(),
        out_type=jax.ShapeDtypeStruct((Sn, D), F32),
        scratch_types=[
            pltpu.VMEM((bpw,), jnp.int32),
            pltpu.VMEM((bpw, D), F32),
            pltpu.SemaphoreType.DMA,
        ],
    )
    def k(idx_hbm, emb_hbm, out_hbm, idx_v, rows_v, sem):
        wid = lax.axis_index("s") * _NC + lax.axis_index("c")
        base = wid * bpw
        pltpu.sync_copy(idx_hbm.at[pl.ds(base, bpw)], idx_v)
        pltpu.async_copy(emb_hbm.at[idx_v], rows_v, sem).wait()
        pltpu.sync_copy(rows_v, out_hbm.at[pl.ds(base, bpw)])

    return k(idx, emb)


def _sc_ce(carr, iarr, labels, V, NQ_):
    (Sn,) = labels.shape
    _, nrow, _ = carr.shape
    npair = NQ_ // 2
    bpw = Sn // _NW
    Vp = ((V + _NL - 1) // _NL) * _NL
    nzero = Vp // _NL
    E = math.e
    R = 2
    nblk = bpw // R
    nck = 128 // _NL

    @functools.partial(
        pl.kernel, mesh=_sc_mesh(),
        compiler_params=pltpu.CompilerParams(needs_layout_passes=False),
        out_type=(jax.ShapeDtypeStruct((Sn, _NL), F32),
                  jax.ShapeDtypeStruct((Sn, _NL), F32)),
        scratch_types=[
            pltpu.VMEM((Vp,), F32),
            pltpu.VMEM((R, nrow, 128), F32),
            pltpu.VMEM((R, nrow, 128), jnp.int32),
            pltpu.VMEM((R, nrow, 128), F32),
            pltpu.VMEM((R, nrow, 128), jnp.int32),
            pltpu.VMEM((bpw,), jnp.int32),
            pltpu.VMEM((bpw, _NL), F32),
            pltpu.VMEM((bpw, _NL), F32),
            pltpu.SemaphoreType.DMA,
            pltpu.SemaphoreType.DMA,
            pltpu.SemaphoreType.DMA,
            pltpu.SemaphoreType.DMA,
        ],
    )
    def k(c_hbm, i_hbm, lab_hbm, part_hbm, alab_hbm,
          accum, vbufa, ibufa, vbufb, ibufb, labv, pout, aout,
          sva, sia, svb, sib):
        wid = lax.axis_index("s") * _NC + lax.axis_index("c")
        base = wid * bpw
        pltpu.sync_copy(lab_hbm.at[pl.ds(base, bpw)], labv)
        zeros16 = jnp.zeros((_NL,), F32)

        def zbody(t, carry):
            accum[pl.ds(t * _NL, _NL)] = zeros16
            return carry

        lax.fori_loop(0, nzero, zbody, 0)

        def fire(t, vbuf, ibuf, sv, si):
            s0 = base + t * R
            pltpu.async_copy(c_hbm.at[pl.ds(s0, R)], vbuf, sv)
            pltpu.async_copy(i_hbm.at[pl.ds(s0, R)], ibuf, si)

        def process(t, vbuf, ibuf, sv, si):
            pltpu.make_async_copy(c_hbm.at[pl.ds(0, R)], vbuf, sv).wait()
            pltpu.make_async_copy(i_hbm.at[pl.ds(0, R)], ibuf, si).wait()
            for r in range(R):
                i = t * R + r

                def pass_a(j, carry):
                    for kk in range(nck):
                        iv = ibuf[r, j, pl.ds(kk * _NL, _NL)]
                        vv = vbuf[r, j, pl.ds(kk * _NL, _NL)]
                        plsc.addupdate_scatter(accum, [iv], vv)
                    return carry

                lax.fori_loop(0, npair, pass_a, 0)

                ivec = jnp.full((_NL,), i, jnp.int32)
                lab = plsc.load_gather(labv, [ivec])
                aout[i] = plsc.load_gather(accum, [lab])

                def pass_b(j, acc):
                    for kk in range(nck):
                        iv = ibuf[r, j, pl.ds(kk * _NL, _NL)]
                        a = plsc.load_gather(accum, [iv])
                        _, lastm = plsc.scan_count(iv)
                        term = jnp.exp(a + 1.0) - E
                        acc = acc + jnp.where(lastm, term, 0.0)
                        plsc.store_scatter(accum, [iv], zeros16)
                    return acc

                acc = lax.fori_loop(0, npair, pass_b, jnp.zeros((_NL,), F32))
                pout[i] = acc

        fire(0, vbufa, ibufa, sva, sia)

        def blk2(u, carry):
            t0 = 2 * u
            fire(t0 + 1, vbufb, ibufb, svb, sib)
            process(t0, vbufa, ibufa, sva, sia)

            @pl.when(t0 + 2 < nblk)
            def _():
                fire(t0 + 2, vbufa, ibufa, sva, sia)

            process(t0 + 1, vbufb, ibufb, svb, sib)
            return carry

        lax.fori_loop(0, nblk // 2, blk2, 0)
        pltpu.sync_copy(pout, part_hbm.at[pl.ds(base, bpw)])
        pltpu.sync_copy(aout, alab_hbm.at[pl.ds(base, bpw)])

    return k(carr, iarr, labels)


def kernel(inputs, response_values, response_indices, emb, gates_w, gates_b, layers):
    B_, S_ = inputs.shape
    V_, D_ = emb.shape
    NQ_, _, _, TK = response_values.shape
    nhead = 2
    nhid = layers[0]["ff1_w"].shape[0]
    nhid_p = 256

    idx = inputs.reshape(S_).astype(jnp.int32)
    x0 = _sc_embed_gather(idx, emb)
    x = _scale_pe(x0, _posenc(S_, D_), math.sqrt(D_))

    for p in layers:
        qkv = _matmul(x, p["in_w"].T, p["in_b"])
        attn = _attention(qkv, S_, D_, nhead)
        proj = _matmul(attn, p["out_w"].T, p["out_b"])
        x = _add_ln(x, proj, p["ln1_w"], p["ln1_b"])
        f1w = jnp.zeros((D_, nhid_p), F32).at[:, :nhid].set(p["ff1_w"].T)
        f1b = jnp.zeros((nhid_p,), F32).at[:nhid].set(p["ff1_b"])
        h = _matmul(x, f1w, f1b, act="relu")
        f2w = jnp.zeros((nhid_p, D_), F32).at[:nhid].set(p["ff2_w"].T)
        f = _matmul(h, f2w, p["ff2_b"])
        x = _add_ln(x, f, p["ln2_w"], p["ln2_b"])

    xl = x[S_ - 1:S_, :]
    score = _matmul(xl, gates_w.T, gates_b, act="sigmoid", mb=1, bf16=False)
    routing_score = score.reshape(-1)

    w2d = _topk(score.reshape(8, -1), NQ_)
    carr, iarr = _contrib(w2d, response_values,
                          response_indices.astype(jnp.int32))
    part, alab = _sc_ce(carr, iarr, idx, V_, NQ_)
    loss = _loss(part, alab, V_)
    return loss.reshape(()), routing_score


# speedup vs baseline: 1.1287x; 1.1287x over previous
import functools
import math

import numpy as np
import jax
import jax.numpy as jnp
from jax import lax
from jax.experimental import pallas as pl
from jax.experimental.pallas import tpu as pltpu
from jax.experimental.pallas import tpu_sc as plsc

F32 = jnp.float32

_NC, _NS, _NL = 2, 16, 16
_NW = _NC * _NS


def _posenc(seq, dim):
    pos = np.arange(seq)[:, None].astype(np.float32)
    div = np.exp(np.arange(0, dim, 2).astype(np.float32) * (-math.log(10000.0) / dim))
    pe = np.zeros((seq, dim), np.float32)
    pe[:, 0::2] = np.sin(pos * div)
    pe[:, 1::2] = np.cos(pos * div)
    return jnp.asarray(pe)


def _ln(x, w, b):
    m = jnp.mean(x, axis=1, keepdims=True)
    var = jnp.mean((x - m) ** 2, axis=1, keepdims=True)
    return (x - m) / jnp.sqrt(var + 1e-5) * w + b


def _bdot(a, b):
    return jnp.dot(a.astype(jnp.bfloat16), b.astype(jnp.bfloat16),
                   preferred_element_type=F32)


def _qkv_body(x_ref, p_ref, w_ref, b_ref, o_ref, *, scale):
    x = x_ref[...]
    if scale is not None:
        x = x * scale + p_ref[...]
    o_ref[...] = _bdot(x, w_ref[...]) + b_ref[...]


def _qkv(x, pe, w, b, scale, mb=256):
    M, D = x.shape
    _, N = w.shape
    args = [x] + ([pe] if scale is not None else []) + [w, b.reshape(1, N)]
    pe_spec = ([pl.BlockSpec((mb, D), lambda i: (i, 0))]
               if scale is not None else [])
    body = (functools.partial(_qkv_body, scale=scale) if scale is not None
            else (lambda x_ref, w_ref, b_ref, o_ref:
                  _qkv_body(x_ref, None, w_ref, b_ref, o_ref, scale=None)))
    return pl.pallas_call(
        body,
        grid=(M // mb,),
        in_specs=[pl.BlockSpec((mb, D), lambda i: (i, 0))] + pe_spec + [
            pl.BlockSpec((D, N), lambda i: (0, 0)),
            pl.BlockSpec((1, N), lambda i: (0, 0)),
        ],
        out_specs=pl.BlockSpec((mb, N), lambda i: (i, 0)),
        out_shape=jax.ShapeDtypeStruct((M, N), F32),
    )(*args)


def _attnln_body(x_ref, p_ref, q_ref, k_ref, v_ref, ow_ref, ob_ref,
                 lw_ref, lb_ref, o_ref, *, sb, S, nhead, dh, scale):
    i = pl.program_id(0)
    rows = lax.broadcasted_iota(jnp.int32, (sb, S), 0) + i * sb
    cols = lax.broadcasted_iota(jnp.int32, (sb, S), 1)
    neg = jnp.float32(-1e30)
    rs = 1.0 / math.sqrt(dh)
    q = q_ref[...]
    heads = []
    for h in range(nhead):
        qh = q[:, h * dh:(h + 1) * dh].astype(jnp.bfloat16)
        kh = k_ref[...][:, h * dh:(h + 1) * dh].astype(jnp.bfloat16)
        vh = v_ref[...][:, h * dh:(h + 1) * dh].astype(jnp.bfloat16)
        s = lax.dot_general(qh, kh, (((1,), (1,)), ((), ())),
                            preferred_element_type=F32) * rs
        s = jnp.where(cols > rows, neg, s)
        m = jnp.max(s, axis=1, keepdims=True)
        p = jnp.exp(s - m)
        p = p / jnp.sum(p, axis=1, keepdims=True)
        heads.append(jnp.dot(p.astype(jnp.bfloat16), vh,
                             preferred_element_type=F32))
    o = jnp.concatenate(heads, axis=1)
    proj = _bdot(o, ow_ref[...]) + ob_ref[...]
    x = x_ref[...]
    if scale is not None:
        x = x * scale + p_ref[...]
    o_ref[...] = _ln(x + proj, lw_ref[...], lb_ref[...])


def _attn_ln(x, pe, qkv, ow, ob, lw, lb, scale, nhead, sb=256):
    S, D = x.shape
    dh = D // nhead
    args = [x] + ([pe] if scale is not None else []) + [
        qkv, qkv, qkv, ow, ob.reshape(1, D), lw.reshape(1, D),
        lb.reshape(1, D)]
    pe_spec = ([pl.BlockSpec((sb, D), lambda i: (i, 0))]
               if scale is not None else [])
    if scale is not None:
        body = functools.partial(_attnln_body, sb=sb, S=S, nhead=nhead,
                                 dh=dh, scale=scale)
    else:
        def body(x_ref, q_ref, k_ref, v_ref, ow_ref, ob_ref, lw_ref,
                 lb_ref, o_ref):
            _attnln_body(x_ref, None, q_ref, k_ref, v_ref, ow_ref, ob_ref,
                         lw_ref, lb_ref, o_ref, sb=sb, S=S, nhead=nhead,
                         dh=dh, scale=None)
    return pl.pallas_call(
        body,
        grid=(S // sb,),
        in_specs=[pl.BlockSpec((sb, D), lambda i: (i, 0))] + pe_spec + [
            pl.BlockSpec((sb, D), lambda i: (i, 0)),
            pl.BlockSpec((S, D), lambda i: (0, 1)),
            pl.BlockSpec((S, D), lambda i: (0, 2)),
            pl.BlockSpec((D, D), lambda i: (0, 0)),
            pl.BlockSpec((1, D), lambda i: (0, 0)),
            pl.BlockSpec((1, D), lambda i: (0, 0)),
            pl.BlockSpec((1, D), lambda i: (0, 0)),
        ],
        out_specs=pl.BlockSpec((sb, D), lambda i: (i, 0)),
        out_shape=jax.ShapeDtypeStruct((S, D), F32),
    )(*args)


def _ffln_body(x_ref, w1_ref, b1_ref, w2_ref, b2_ref, lw_ref, lb_ref, o_ref):
    x = x_ref[...]
    h = jnp.maximum(_bdot(x, w1_ref[...]) + b1_ref[...], 0.0)
    f = _bdot(h, w2_ref[...]) + b2_ref[...]
    o_ref[...] = _ln(x + f, lw_ref[...], lb_ref[...])


def _ff_ln(x, w1, b1, w2, b2, lw, lb, mb=256):
    M, D = x.shape
    _, H = w1.shape
    return pl.pallas_call(
        _ffln_body,
        grid=(M // mb,),
        in_specs=[
            pl.BlockSpec((mb, D), lambda i: (i, 0)),
            pl.BlockSpec((D, H), lambda i: (0, 0)),
            pl.BlockSpec((1, H), lambda i: (0, 0)),
            pl.BlockSpec((H, D), lambda i: (0, 0)),
            pl.BlockSpec((1, D), lambda i: (0, 0)),
            pl.BlockSpec((1, D), lambda i: (0, 0)),
            pl.BlockSpec((1, D), lambda i: (0, 0)),
        ],
        out_specs=pl.BlockSpec((mb, D), lambda i: (i, 0)),
        out_shape=jax.ShapeDtypeStruct((M, D), F32),
    )(x, w1, b1.reshape(1, H), w2, b2.reshape(1, D), lw.reshape(1, D),
      lb.reshape(1, D))


def _gatetopk_body(x_ref, w_ref, b_ref, s_ref, o_ref, *, nq, mb):
    xl = x_ref[mb - 1:mb, :]
    sc = jnp.dot(xl, w_ref[...], preferred_element_type=F32) + b_ref[...]
    sc = 1.0 / (1.0 + jnp.exp(-sc))
    s_ref[...] = sc
    R, C = sc.shape
    flat = (lax.broadcasted_iota(jnp.int32, (R, C), 0) * C
            + lax.broadcasted_iota(jnp.int32, (R, C), 1))
    rowi = lax.broadcasted_iota(jnp.int32, (64, 128), 0)
    coli = lax.broadcasted_iota(jnp.int32, (64, 128), 1)

    def body(t, carry):
        s, o = carry
        m = jnp.max(s)
        cand = jnp.where(s == m, flat, jnp.int32(2 ** 30))
        amin = jnp.min(cand)
        o = o + jnp.where(rowi == t, m, 0.0)
        s = jnp.where(flat == amin, jnp.float32(-1e30), s)
        return s, o

    sc, o = lax.fori_loop(0, nq, body, (sc, jnp.zeros((64, 128), F32)))
    total = jnp.sum(jnp.where(coli == 0, o, 0.0))
    o_ref[...] = o / total


def _gate_topk(x, gw, gb, nq, mb=256):
    S, D = x.shape
    _, NG = gw.shape
    return pl.pallas_call(
        functools.partial(_gatetopk_body, nq=nq, mb=mb),
        grid=(1,),
        in_specs=[
            pl.BlockSpec((mb, D), lambda i: (S // mb - 1, 0)),
            pl.BlockSpec((D, NG), lambda i: (0, 0)),
            pl.BlockSpec((1, NG), lambda i: (0, 0)),
        ],
        out_specs=(pl.BlockSpec((1, NG), lambda i: (0, 0)),
                   pl.BlockSpec((64, 128), lambda i: (0, 0))),
        out_shape=(jax.ShapeDtypeStruct((1, NG), F32),
                   jax.ShapeDtypeStruct((64, 128), F32)),
    )(x, gw, gb.reshape(1, NG))


def _contrib_body(w_ref, rv_ref, ri_ref, oc_ref, oi_ref, *, sb, npair, nrow):
    rv = rv_ref[...]
    ri = ri_ref[...]
    cps = []
    ips = []
    for j in range(npair):
        a = jnp.log(w_ref[2 * j, 0, 0] * rv[2 * j, 0] + 1e-40)
        b = jnp.log(w_ref[2 * j + 1, 0, 0] * rv[2 * j + 1, 0] + 1e-40)
        cps.append(jnp.concatenate([a, b], axis=1)[:, None, :])
        ips.append(jnp.concatenate([ri[2 * j, 0], ri[2 * j + 1, 0]],
                                   axis=1)[:, None, :])
    pad = nrow - npair
    cps.append(jnp.zeros((sb, pad, 128), F32))
    ips.append(jnp.zeros((sb, pad, 128), jnp.int32))
    oc_ref[...] = jnp.concatenate(cps, axis=1)
    oi_ref[...] = jnp.concatenate(ips, axis=1)


def _contrib(w2d, rv4, ri4, sb=128, nrow=32):
    NQ_, _, S_, TK = rv4.shape
    npair = NQ_ // 2
    w3 = w2d.reshape(64, 1, 128)
    return pl.pallas_call(
        functools.partial(_contrib_body, sb=sb, npair=npair, nrow=nrow),
        grid=(S_ // sb,),
        in_specs=[
            pl.BlockSpec((64, 1, 128), lambda s: (0, 0, 0)),
            pl.BlockSpec((NQ_, 1, sb, TK), lambda s: (0, 0, s, 0)),
            pl.BlockSpec((NQ_, 1, sb, TK), lambda s: (0, 0, s, 0)),
        ],
        out_specs=(pl.BlockSpec((sb, nrow, 128), lambda s: (s, 0, 0)),
                   pl.BlockSpec((sb, nrow, 128), lambda s: (s, 0, 0))),
        out_shape=(jax.ShapeDtypeStruct((S_, nrow, 128), F32),
                   jax.ShapeDtypeStruct((S_, nrow, 128), jnp.int32)),
    )(w3, rv4, ri4)


def _loss_body(p_ref, a_ref, o_ref, *, V):
    part = jnp.sum(p_ref[...], axis=1, keepdims=True)
    alab = a_ref[...][:, 0:1]
    lr = jnp.log(V * math.e + part) - 1.0 - alab
    o_ref[...] = jnp.mean(lr).reshape(1, 1)


def _loss(part, alab, V):
    return pl.pallas_call(
        functools.partial(_loss_body, V=V),
        out_shape=jax.ShapeDtypeStruct((1, 1), F32),
    )(part, alab)


def _sc_mesh():
    return plsc.VectorSubcoreMesh(
        core_axis_name="c", subcore_axis_name="s",
        num_cores=_NC, num_subcores=_NS)


def _sc_embed_gather(idx, emb):
    (Sn,) = idx.shape
    V, D = emb.shape
    bpw = Sn // _NW

    @functools.partial(
        pl.kernel, mesh=_sc_mesh(),
        out_type=jax.ShapeDtypeStruct((Sn, D), F32),
        scratch_types=[
            pltpu.VMEM((bpw,), jnp.int32),
            pltpu.VMEM((bpw, D), F32),
            pltpu.SemaphoreType.DMA,
        ],
    )
    def k(idx_hbm, emb_hbm, out_hbm, idx_v, rows_v, sem):
        wid = lax.axis_index("s") * _NC + lax.axis_index("c")
        base = wid * bpw
        pltpu.sync_copy(idx_hbm.at[pl.ds(base, bpw)], idx_v)
        pltpu.async_copy(emb_hbm.at[idx_v], rows_v, sem).wait()
        pltpu.sync_copy(rows_v, out_hbm.at[pl.ds(base, bpw)])

    return k(idx, emb)


def _sc_ce(carr, iarr, labels, V, NQ_):
    (Sn,) = labels.shape
    _, nrow, _ = carr.shape
    npair = NQ_ // 2
    bpw = Sn // _NW
    Vp = ((V + _NL - 1) // _NL) * _NL
    nzero = Vp // _NL
    E = math.e
    R = 2
    nblk = bpw // R
    nck = 128 // _NL

    @functools.partial(
        pl.kernel, mesh=_sc_mesh(),
        compiler_params=pltpu.CompilerParams(needs_layout_passes=False),
        out_type=(jax.ShapeDtypeStruct((Sn, _NL), F32),
                  jax.ShapeDtypeStruct((Sn, _NL), F32)),
        scratch_types=[
            pltpu.VMEM((Vp,), F32),
            pltpu.VMEM((R, nrow, 128), F32),
            pltpu.VMEM((R, nrow, 128), jnp.int32),
            pltpu.VMEM((R, nrow, 128), F32),
            pltpu.VMEM((R, nrow, 128), jnp.int32),
            pltpu.VMEM((bpw,), jnp.int32),
            pltpu.VMEM((bpw, _NL), F32),
            pltpu.VMEM((bpw, _NL), F32),
            pltpu.SemaphoreType.DMA,
            pltpu.SemaphoreType.DMA,
            pltpu.SemaphoreType.DMA,
            pltpu.SemaphoreType.DMA,
        ],
    )
    def k(c_hbm, i_hbm, lab_hbm, part_hbm, alab_hbm,
          accum, vbufa, ibufa, vbufb, ibufb, labv, pout, aout,
          sva, sia, svb, sib):
        wid = lax.axis_index("s") * _NC + lax.axis_index("c")
        base = wid * bpw
        pltpu.sync_copy(lab_hbm.at[pl.ds(base, bpw)], labv)
        zeros16 = jnp.zeros((_NL,), F32)

        def zbody(t, carry):
            accum[pl.ds(t * _NL, _NL)] = zeros16
            return carry

        lax.fori_loop(0, nzero, zbody, 0)

        def fire(t, vbuf, ibuf, sv, si):
            s0 = base + t * R
            pltpu.async_copy(c_hbm.at[pl.ds(s0, R)], vbuf, sv)
            pltpu.async_copy(i_hbm.at[pl.ds(s0, R)], ibuf, si)

        def process(t, vbuf, ibuf, sv, si):
            pltpu.make_async_copy(c_hbm.at[pl.ds(0, R)], vbuf, sv).wait()
            pltpu.make_async_copy(i_hbm.at[pl.ds(0, R)], ibuf, si).wait()
            for r in range(R):
                i = t * R + r

                def pass_a(j, carry):
                    for kk in range(nck):
                        iv = ibuf[r, j, pl.ds(kk * _NL, _NL)]
                        vv = vbuf[r, j, pl.ds(kk * _NL, _NL)]
                        plsc.addupdate_scatter(accum, [iv], vv)
                    return carry

                lax.fori_loop(0, npair, pass_a, 0)

                ivec = jnp.full((_NL,), i, jnp.int32)
                lab = plsc.load_gather(labv, [ivec])
                aout[i] = plsc.load_gather(accum, [lab])

                def pass_b(j, acc):
                    for kk in range(nck):
                        iv = ibuf[r, j, pl.ds(kk * _NL, _NL)]
                        a = plsc.load_gather(accum, [iv])
                        _, lastm = plsc.scan_count(iv)
                        term = jnp.exp(a + 1.0) - E
                        acc = acc + jnp.where(lastm, term, 0.0)
                        plsc.store_scatter(accum, [iv], zeros16)
                    return acc

                acc = lax.fori_loop(0, npair, pass_b, jnp.zeros((_NL,), F32))
                pout[i] = acc

        fire(0, vbufa, ibufa, sva, sia)

        def blk2(u, carry):
            t0 = 2 * u
            fire(t0 + 1, vbufb, ibufb, svb, sib)
            process(t0, vbufa, ibufa, sva, sia)

            @pl.when(t0 + 2 < nblk)
            def _():
                fire(t0 + 2, vbufa, ibufa, sva, sia)

            process(t0 + 1, vbufb, ibufb, svb, sib)
            return carry

        lax.fori_loop(0, nblk // 2, blk2, 0)
        pltpu.sync_copy(pout, part_hbm.at[pl.ds(base, bpw)])
        pltpu.sync_copy(aout, alab_hbm.at[pl.ds(base, bpw)])

    return k(carr, iarr, labels)


def kernel(inputs, response_values, response_indices, emb, gates_w, gates_b, layers):
    B_, S_ = inputs.shape
    V_, D_ = emb.shape
    NQ_, _, _, TK = response_values.shape
    nhead = 2
    nhid = layers[0]["ff1_w"].shape[0]
    nhid_p = 256

    idx = inputs.reshape(S_).astype(jnp.int32)
    x = _sc_embed_gather(idx, emb)
    pe = _posenc(S_, D_)
    scale = math.sqrt(D_)

    for li, p in enumerate(layers):
        sc = scale if li == 0 else None
        qkv = _qkv(x, pe, p["in_w"].T, p["in_b"], sc)
        x1 = _attn_ln(x, pe, qkv, p["out_w"].T, p["out_b"],
                      p["ln1_w"], p["ln1_b"], sc, nhead)
        f1w = jnp.zeros((D_, nhid_p), F32).at[:, :nhid].set(p["ff1_w"].T)
        f1b = jnp.zeros((nhid_p,), F32).at[:nhid].set(p["ff1_b"])
        f2w = jnp.zeros((nhid_p, D_), F32).at[:nhid].set(p["ff2_w"].T)
        x = _ff_ln(x1, f1w, f1b, f2w, p["ff2_b"], p["ln2_w"], p["ln2_b"])

    score, w2d = _gate_topk(x, gates_w.T, gates_b, NQ_)
    routing_score = score.reshape(-1)
    carr, iarr = _contrib(w2d, response_values,
                          response_indices.astype(jnp.int32))
    part, alab = _sc_ce(carr, iarr, idx, V_, NQ_)
    loss = _loss(part, alab, V_)
    return loss.reshape(()), routing_score


# speedup vs baseline: 1.2863x; 1.1396x over previous
import functools
import math

import numpy as np
import jax
import jax.numpy as jnp
from jax import lax
from jax.experimental import pallas as pl
from jax.experimental.pallas import tpu as pltpu
from jax.experimental.pallas import tpu_sc as plsc

F32 = jnp.float32

_NC, _NS, _NL = 2, 16, 16
_NW = _NC * _NS


def _posenc(seq, dim):
    pos = np.arange(seq)[:, None].astype(np.float32)
    div = np.exp(np.arange(0, dim, 2).astype(np.float32) * (-math.log(10000.0) / dim))
    pe = np.zeros((seq, dim), np.float32)
    pe[:, 0::2] = np.sin(pos * div)
    pe[:, 1::2] = np.cos(pos * div)
    return jnp.asarray(pe)


def _ln(x, w, b):
    m = jnp.mean(x, axis=1, keepdims=True)
    var = jnp.mean((x - m) ** 2, axis=1, keepdims=True)
    return (x - m) / jnp.sqrt(var + 1e-5) * w + b


def _bdot_t(a, b):
    return lax.dot_general(a.astype(jnp.bfloat16), b.astype(jnp.bfloat16),
                           (((1,), (1,)), ((), ())),
                           preferred_element_type=F32)


def _qkv_body(x_ref, p_ref, w_ref, b_ref, o_ref, *, scale):
    x = x_ref[...]
    if scale is not None:
        x = x * scale + p_ref[...]
    o_ref[...] = _bdot_t(x, w_ref[...]) + b_ref[...]


def _qkv(x, pe, w, b, scale, mb=256):
    M, D = x.shape
    N, _ = w.shape
    args = [x] + ([pe] if scale is not None else []) + [w, b.reshape(1, N)]
    pe_spec = ([pl.BlockSpec((mb, D), lambda i: (i, 0))]
               if scale is not None else [])
    body = (functools.partial(_qkv_body, scale=scale) if scale is not None
            else (lambda x_ref, w_ref, b_ref, o_ref:
                  _qkv_body(x_ref, None, w_ref, b_ref, o_ref, scale=None)))
    return pl.pallas_call(
        body,
        grid=(M // mb,),
        in_specs=[pl.BlockSpec((mb, D), lambda i: (i, 0))] + pe_spec + [
            pl.BlockSpec((N, D), lambda i: (0, 0)),
            pl.BlockSpec((1, N), lambda i: (0, 0)),
        ],
        out_specs=pl.BlockSpec((mb, N), lambda i: (i, 0)),
        out_shape=jax.ShapeDtypeStruct((M, N), F32),
    )(*args)


def _attnln_body(x_ref, p_ref, q_ref, k_ref, v_ref, ow_ref, ob_ref,
                 lw_ref, lb_ref, o_ref, *, sb, S, nhead, dh, scale,
                 last_only):
    if last_only:
        row0 = S - sb
    else:
        row0 = pl.program_id(0) * sb
    rows = lax.broadcasted_iota(jnp.int32, (sb, S), 0) + row0
    cols = lax.broadcasted_iota(jnp.int32, (sb, S), 1)
    neg = jnp.float32(-1e30)
    rs = 1.0 / math.sqrt(dh)
    q = q_ref[...]
    heads = []
    for h in range(nhead):
        qh = q[:, h * dh:(h + 1) * dh].astype(jnp.bfloat16)
        kh = k_ref[...][:, h * dh:(h + 1) * dh].astype(jnp.bfloat16)
        vh = v_ref[...][:, h * dh:(h + 1) * dh].astype(jnp.bfloat16)
        s = lax.dot_general(qh, kh, (((1,), (1,)), ((), ())),
                            preferred_element_type=F32) * rs
        s = jnp.where(cols > rows, neg, s)
        m = jnp.max(s, axis=1, keepdims=True)
        p = jnp.exp(s - m)
        p = p / jnp.sum(p, axis=1, keepdims=True)
        heads.append(jnp.dot(p.astype(jnp.bfloat16), vh,
                             preferred_element_type=F32))
    o = jnp.concatenate(heads, axis=1)
    proj = _bdot_t(o, ow_ref[...]) + ob_ref[...]
    x = x_ref[...]
    if scale is not None:
        x = x * scale + p_ref[...]
    o_ref[...] = _ln(x + proj, lw_ref[...], lb_ref[...])


def _attn_ln(x, pe, qkv, ow, ob, lw, lb, scale, nhead, last_only=False,
             sb=256):
    S, D = x.shape
    dh = D // nhead
    nb = S // sb
    last = nb - 1
    xmap = (lambda i: (last, 0)) if last_only else (lambda i: (i, 0))
    args = [x] + ([pe] if scale is not None else []) + [
        qkv, qkv, qkv, ow, ob.reshape(1, D), lw.reshape(1, D),
        lb.reshape(1, D)]
    pe_spec = ([pl.BlockSpec((sb, D), xmap)] if scale is not None else [])
    kw = dict(sb=sb, S=S, nhead=nhead, dh=dh, scale=scale,
              last_only=last_only)
    if scale is not None:
        body = functools.partial(_attnln_body, **kw)
    else:
        def body(x_ref, q_ref, k_ref, v_ref, ow_ref, ob_ref, lw_ref,
                 lb_ref, o_ref):
            _attnln_body(x_ref, None, q_ref, k_ref, v_ref, ow_ref, ob_ref,
                         lw_ref, lb_ref, o_ref, **kw)
    return pl.pallas_call(
        body,
        grid=(1 if last_only else nb,),
        in_specs=[pl.BlockSpec((sb, D), xmap)] + pe_spec + [
            pl.BlockSpec((sb, D), xmap),
            pl.BlockSpec((S, D), lambda i: (0, 1)),
            pl.BlockSpec((S, D), lambda i: (0, 2)),
            pl.BlockSpec((D, D), lambda i: (0, 0)),
            pl.BlockSpec((1, D), lambda i: (0, 0)),
            pl.BlockSpec((1, D), lambda i: (0, 0)),
            pl.BlockSpec((1, D), lambda i: (0, 0)),
        ],
        out_specs=pl.BlockSpec((sb, D), (lambda i: (0, 0)) if last_only
                               else (lambda i: (i, 0))),
        out_shape=jax.ShapeDtypeStruct((sb if last_only else S, D), F32),
    )(*args)


def _ff(x, w1_ref, b1_ref, w2_ref, b2_ref):
    h = jnp.maximum(_bdot_t(x, w1_ref[...]) + b1_ref[...], 0.0)
    return _bdot_t(h, w2_ref[...]) + b2_ref[...]


def _ffln_body(x_ref, w1_ref, b1_ref, w2_ref, b2_ref, lw_ref, lb_ref, o_ref):
    x = x_ref[...]
    f = _ff(x, w1_ref, b1_ref, w2_ref, b2_ref)
    o_ref[...] = _ln(x + f, lw_ref[...], lb_ref[...])


def _ffln_specs(D, H, NG=None):
    sp = [
        pl.BlockSpec((H, D), lambda i: (0, 0)),
        pl.BlockSpec((1, H), lambda i: (0, 0)),
        pl.BlockSpec((D, H), lambda i: (0, 0)),
        pl.BlockSpec((1, D), lambda i: (0, 0)),
        pl.BlockSpec((1, D), lambda i: (0, 0)),
        pl.BlockSpec((1, D), lambda i: (0, 0)),
    ]
    if NG is not None:
        sp += [pl.BlockSpec((NG, D), lambda i: (0, 0)),
               pl.BlockSpec((1, NG), lambda i: (0, 0))]
    return sp


def _ff_ln(x, w1, b1, w2, b2, lw, lb, mb=256):
    M, D = x.shape
    H, _ = w1.shape
    return pl.pallas_call(
        _ffln_body,
        grid=(M // mb,),
        in_specs=[pl.BlockSpec((mb, D), lambda i: (i, 0))] + _ffln_specs(D, H),
        out_specs=pl.BlockSpec((mb, D), lambda i: (i, 0)),
        out_shape=jax.ShapeDtypeStruct((M, D), F32),
    )(x, w1, b1.reshape(1, H), w2, b2.reshape(1, D), lw.reshape(1, D),
      lb.reshape(1, D))


def _fflngate_body(x_ref, w1_ref, b1_ref, w2_ref, b2_ref, lw_ref, lb_ref,
                   gw_ref, gb_ref, s_ref, o_ref, *, nq, mb):
    x = x_ref[...]
    f = _ff(x, w1_ref, b1_ref, w2_ref, b2_ref)
    x2 = _ln(x + f, lw_ref[...], lb_ref[...])
    xl = x2[mb - 1:mb, :]
    sc = lax.dot_general(xl, gw_ref[...], (((1,), (1,)), ((), ())),
                         preferred_element_type=F32) + gb_ref[...]
    sc = 1.0 / (1.0 + jnp.exp(-sc))
    s_ref[...] = sc
    R, C = sc.shape
    flat = (lax.broadcasted_iota(jnp.int32, (R, C), 0) * C
            + lax.broadcasted_iota(jnp.int32, (R, C), 1))
    rowi = lax.broadcasted_iota(jnp.int32, (64, 128), 0)
    coli = lax.broadcasted_iota(jnp.int32, (64, 128), 1)

    def body(t, carry):
        s, o = carry
        m = jnp.max(s)
        cand = jnp.where(s == m, flat, jnp.int32(2 ** 30))
        amin = jnp.min(cand)
        o = o + jnp.where(rowi == t, m, 0.0)
        s = jnp.where(flat == amin, jnp.float32(-1e30), s)
        return s, o

    sc, o = lax.fori_loop(0, nq, body, (sc, jnp.zeros((64, 128), F32)))
    total = jnp.sum(jnp.where(coli == 0, o, 0.0))
    o_ref[...] = o / total


def _ff_ln_gate(xlast, w1, b1, w2, b2, lw, lb, gw, gb, nq):
    mb, D = xlast.shape
    H, _ = w1.shape
    NG, _ = gw.shape
    return pl.pallas_call(
        functools.partial(_fflngate_body, nq=nq, mb=mb),
        grid=(1,),
        in_specs=[pl.BlockSpec((mb, D), lambda i: (0, 0))]
        + _ffln_specs(D, H, NG),
        out_specs=(pl.BlockSpec((1, NG), lambda i: (0, 0)),
                   pl.BlockSpec((64, 128), lambda i: (0, 0))),
        out_shape=(jax.ShapeDtypeStruct((1, NG), F32),
                   jax.ShapeDtypeStruct((64, 128), F32)),
    )(xlast, w1, b1.reshape(1, H), w2, b2.reshape(1, D), lw.reshape(1, D),
      lb.reshape(1, D), gw, gb.reshape(1, NG))


def _contrib_body(w_ref, rv_ref, ri_ref, oc_ref, oi_ref, *, sb, npair, nrow):
    rv = rv_ref[...]
    ri = ri_ref[...]
    cps = []
    ips = []
    for j in range(npair):
        a = jnp.log(w_ref[2 * j, 0, 0] * rv[2 * j, 0] + 1e-40)
        b = jnp.log(w_ref[2 * j + 1, 0, 0] * rv[2 * j + 1, 0] + 1e-40)
        cps.append(jnp.concatenate([a, b], axis=1)[:, None, :])
        ips.append(jnp.concatenate([ri[2 * j, 0], ri[2 * j + 1, 0]],
                                   axis=1)[:, None, :])
    pad = nrow - npair
    cps.append(jnp.zeros((sb, pad, 128), F32))
    ips.append(jnp.zeros((sb, pad, 128), jnp.int32))
    oc_ref[...] = jnp.concatenate(cps, axis=1)
    oi_ref[...] = jnp.concatenate(ips, axis=1)


def _contrib(w2d, rv4, ri4, sb=128, nrow=32):
    NQ_, _, S_, TK = rv4.shape
    npair = NQ_ // 2
    w3 = w2d.reshape(64, 1, 128)
    return pl.pallas_call(
        functools.partial(_contrib_body, sb=sb, npair=npair, nrow=nrow),
        grid=(S_ // sb,),
        in_specs=[
            pl.BlockSpec((64, 1, 128), lambda s: (0, 0, 0)),
            pl.BlockSpec((NQ_, 1, sb, TK), lambda s: (0, 0, s, 0)),
            pl.BlockSpec((NQ_, 1, sb, TK), lambda s: (0, 0, s, 0)),
        ],
        out_specs=(pl.BlockSpec((sb, nrow, 128), lambda s: (s, 0, 0)),
                   pl.BlockSpec((sb, nrow, 128), lambda s: (s, 0, 0))),
        out_shape=(jax.ShapeDtypeStruct((S_, nrow, 128), F32),
                   jax.ShapeDtypeStruct((S_, nrow, 128), jnp.int32)),
    )(w3, rv4, ri4)


def _loss_body(p_ref, a_ref, o_ref, *, V):
    part = jnp.sum(p_ref[...], axis=1, keepdims=True)
    alab = a_ref[...][:, 0:1]
    lr = jnp.log(V * math.e + part) - 1.0 - alab
    o_ref[...] = jnp.mean(lr).reshape(1, 1)


def _loss(part, alab, V):
    return pl.pallas_call(
        functools.partial(_loss_body, V=V),
        out_shape=jax.ShapeDtypeStruct((1, 1), F32),
    )(part, alab)


def _sc_mesh():
    return plsc.VectorSubcoreMesh(
        core_axis_name="c", subcore_axis_name="s",
        num_cores=_NC, num_subcores=_NS)


def _sc_embed_gather(idx, emb):
    (Sn,) = idx.shape
    V, D = emb.shape
    bpw = Sn // _NW

    @functools.partial(
        pl.kernel, mesh=_sc_mesh(),
        out_type=jax.ShapeDtypeStruct((Sn, D), F32),
        scratch_types=[
            pltpu.VMEM((bpw,), jnp.int32),
            pltpu.VMEM((bpw, D), F32),
            pltpu.SemaphoreType.DMA,
        ],
    )
    def k(idx_hbm, emb_hbm, out_hbm, idx_v, rows_v, sem):
        wid = lax.axis_index("s") * _NC + lax.axis_index("c")
        base = wid * bpw
        pltpu.sync_copy(idx_hbm.at[pl.ds(base, bpw)], idx_v)
        pltpu.async_copy(emb_hbm.at[idx_v], rows_v, sem).wait()
        pltpu.sync_copy(rows_v, out_hbm.at[pl.ds(base, bpw)])

    return k(idx, emb)


def _sc_ce(carr, iarr, labels, V, NQ_):
    (Sn,) = labels.shape
    _, nrow, _ = carr.shape
    npair = NQ_ // 2
    bpw = Sn // _NW
    Vp = ((V + _NL - 1) // _NL) * _NL
    nzero = Vp // _NL
    E = math.e
    R = 2
    nblk = bpw // R
    nck = 128 // _NL

    @functools.partial(
        pl.kernel, mesh=_sc_mesh(),
        compiler_params=pltpu.CompilerParams(needs_layout_passes=False),
        out_type=(jax.ShapeDtypeStruct((Sn, _NL), F32),
                  jax.ShapeDtypeStruct((Sn, _NL), F32)),
        scratch_types=[
            pltpu.VMEM((Vp,), F32),
            pltpu.VMEM((R, nrow, 128), F32),
            pltpu.VMEM((R, nrow, 128), jnp.int32),
            pltpu.VMEM((R, nrow, 128), F32),
            pltpu.VMEM((R, nrow, 128), jnp.int32),
            pltpu.VMEM((bpw,), jnp.int32),
            pltpu.VMEM((bpw, _NL), F32),
            pltpu.VMEM((bpw, _NL), F32),
            pltpu.SemaphoreType.DMA,
            pltpu.SemaphoreType.DMA,
            pltpu.SemaphoreType.DMA,
            pltpu.SemaphoreType.DMA,
        ],
    )
    def k(c_hbm, i_hbm, lab_hbm, part_hbm, alab_hbm,
          accum, vbufa, ibufa, vbufb, ibufb, labv, pout, aout,
          sva, sia, svb, sib):
        wid = lax.axis_index("s") * _NC + lax.axis_index("c")
        base = wid * bpw
        pltpu.sync_copy(lab_hbm.at[pl.ds(base, bpw)], labv)
        zeros16 = jnp.zeros((_NL,), F32)

        def zbody(t, carry):
            accum[pl.ds(t * _NL, _NL)] = zeros16
            return carry

        lax.fori_loop(0, nzero, zbody, 0)

        def fire(t, vbuf, ibuf, sv, si):
            s0 = base + t * R
            pltpu.async_copy(c_hbm.at[pl.ds(s0, R)], vbuf, sv)
            pltpu.async_copy(i_hbm.at[pl.ds(s0, R)], ibuf, si)

        def process(t, vbuf, ibuf, sv, si):
            pltpu.make_async_copy(c_hbm.at[pl.ds(0, R)], vbuf, sv).wait()
            pltpu.make_async_copy(i_hbm.at[pl.ds(0, R)], ibuf, si).wait()
            for r in range(R):
                i = t * R + r

                def pass_a(j, carry):
                    for kk in range(nck):
                        iv = ibuf[r, j, pl.ds(kk * _NL, _NL)]
                        vv = vbuf[r, j, pl.ds(kk * _NL, _NL)]
                        plsc.addupdate_scatter(accum, [iv], vv)
                    return carry

                lax.fori_loop(0, npair, pass_a, 0)

                ivec = jnp.full((_NL,), i, jnp.int32)
                lab = plsc.load_gather(labv, [ivec])
                aout[i] = plsc.load_gather(accum, [lab])

                def pass_b(j, acc):
                    for kk in range(nck):
                        iv = ibuf[r, j, pl.ds(kk * _NL, _NL)]
                        a = plsc.load_gather(accum, [iv])
                        _, lastm = plsc.scan_count(iv)
                        term = jnp.exp(a + 1.0) - E
                        acc = acc + jnp.where(lastm, term, 0.0)
                        plsc.store_scatter(accum, [iv], zeros16)
                    return acc

                acc = lax.fori_loop(0, npair, pass_b, jnp.zeros((_NL,), F32))
                pout[i] = acc

        fire(0, vbufa, ibufa, sva, sia)

        def blk2(u, carry):
            t0 = 2 * u
            fire(t0 + 1, vbufb, ibufb, svb, sib)
            process(t0, vbufa, ibufa, sva, sia)

            @pl.when(t0 + 2 < nblk)
            def _():
                fire(t0 + 2, vbufa, ibufa, sva, sia)

            process(t0 + 1, vbufb, ibufb, svb, sib)
            return carry

        lax.fori_loop(0, nblk // 2, blk2, 0)
        pltpu.sync_copy(pout, part_hbm.at[pl.ds(base, bpw)])
        pltpu.sync_copy(aout, alab_hbm.at[pl.ds(base, bpw)])

    return k(carr, iarr, labels)


def kernel(inputs, response_values, response_indices, emb, gates_w, gates_b, layers):
    B_, S_ = inputs.shape
    V_, D_ = emb.shape
    NQ_, _, _, TK = response_values.shape
    nhead = 2
    nhid = layers[0]["ff1_w"].shape[0]
    nhid_p = 256

    idx = inputs.reshape(S_).astype(jnp.int32)
    x = _sc_embed_gather(idx, emb)
    pe = _posenc(S_, D_)
    scale = math.sqrt(D_)

    nl = len(layers)
    for li, p in enumerate(layers):
        sc = scale if li == 0 else None
        last = li == nl - 1
        qkv = _qkv(x, pe, p["in_w"], p["in_b"], sc)
        x1 = _attn_ln(x, pe, qkv, p["out_w"], p["out_b"],
                      p["ln1_w"], p["ln1_b"], sc, nhead, last_only=last)
        f1w = jnp.zeros((nhid_p, D_), F32).at[:nhid].set(p["ff1_w"])
        f1b = jnp.zeros((nhid_p,), F32).at[:nhid].set(p["ff1_b"])
        f2w = jnp.zeros((D_, nhid_p), F32).at[:, :nhid].set(p["ff2_w"])
        if last:
            score, w2d = _ff_ln_gate(x1, f1w, f1b, f2w, p["ff2_b"],
                                     p["ln2_w"], p["ln2_b"],
                                     gates_w, gates_b, NQ_)
        else:
            x = _ff_ln(x1, f1w, f1b, f2w, p["ff2_b"], p["ln2_w"], p["ln2_b"])

    routing_score = score.reshape(-1)
    carr, iarr = _contrib(w2d, response_values,
                          response_indices.astype(jnp.int32))
    part, alab = _sc_ce(carr, iarr, idx, V_, NQ_)
    loss = _loss(part, alab, V_)
    return loss.reshape(()), routing_score


# speedup vs baseline: 1.3672x; 1.0629x over previous
import functools
import math

import numpy as np
import jax
import jax.numpy as jnp
from jax import lax
from jax.experimental import pallas as pl
from jax.experimental.pallas import tpu as pltpu
from jax.experimental.pallas import tpu_sc as plsc

F32 = jnp.float32

_NC, _NS, _NL = 2, 16, 16
_NW = _NC * _NS


def _posenc(seq, dim):
    pos = np.arange(seq)[:, None].astype(np.float32)
    div = np.exp(np.arange(0, dim, 2).astype(np.float32) * (-math.log(10000.0) / dim))
    pe = np.zeros((seq, dim), np.float32)
    pe[:, 0::2] = np.sin(pos * div)
    pe[:, 1::2] = np.cos(pos * div)
    return jnp.asarray(pe)


def _ln(x, w, b):
    m = jnp.mean(x, axis=1, keepdims=True)
    var = jnp.mean((x - m) ** 2, axis=1, keepdims=True)
    return (x - m) / jnp.sqrt(var + 1e-5) * w + b


def _bdot_t(a, b):
    return lax.dot_general(a.astype(jnp.bfloat16), b.astype(jnp.bfloat16),
                           (((1,), (1,)), ((), ())),
                           preferred_element_type=F32)


def _qkv_body(x_ref, p_ref, w_ref, b_ref, o_ref, *, scale):
    x = x_ref[...]
    if scale is not None:
        x = x * scale + p_ref[...]
    o_ref[...] = _bdot_t(x, w_ref[...]) + b_ref[...]


def _qkv(x, pe, w, b, scale, mb=256):
    M, D = x.shape
    N, _ = w.shape
    args = [x] + ([pe] if scale is not None else []) + [w, b.reshape(1, N)]
    pe_spec = ([pl.BlockSpec((mb, D), lambda i: (i, 0))]
               if scale is not None else [])
    body = (functools.partial(_qkv_body, scale=scale) if scale is not None
            else (lambda x_ref, w_ref, b_ref, o_ref:
                  _qkv_body(x_ref, None, w_ref, b_ref, o_ref, scale=None)))
    return pl.pallas_call(
        body,
        grid=(M // mb,),
        in_specs=[pl.BlockSpec((mb, D), lambda i: (i, 0))] + pe_spec + [
            pl.BlockSpec((N, D), lambda i: (0, 0)),
            pl.BlockSpec((1, N), lambda i: (0, 0)),
        ],
        out_specs=pl.BlockSpec((mb, N), lambda i: (i, 0)),
        out_shape=jax.ShapeDtypeStruct((M, N), F32),
    )(*args)


def _attnln_body(x_ref, p_ref, q_ref, k_ref, v_ref, ow_ref, ob_ref,
                 lw_ref, lb_ref, o_ref, *, sb, S, nhead, dh, scale,
                 last_only):
    if last_only:
        row0 = S - sb
    else:
        row0 = pl.program_id(0) * sb
    rows = lax.broadcasted_iota(jnp.int32, (sb, S), 0) + row0
    cols = lax.broadcasted_iota(jnp.int32, (sb, S), 1)
    neg = jnp.float32(-1e30)
    rs = 1.0 / math.sqrt(dh)
    q = q_ref[...]
    heads = []
    for h in range(nhead):
        qh = q[:, h * dh:(h + 1) * dh].astype(jnp.bfloat16)
        kh = k_ref[...][:, h * dh:(h + 1) * dh].astype(jnp.bfloat16)
        vh = v_ref[...][:, h * dh:(h + 1) * dh].astype(jnp.bfloat16)
        s = lax.dot_general(qh, kh, (((1,), (1,)), ((), ())),
                            preferred_element_type=F32) * rs
        s = jnp.where(cols > rows, neg, s)
        m = jnp.max(s, axis=1, keepdims=True)
        p = jnp.exp(s - m)
        p = p / jnp.sum(p, axis=1, keepdims=True)
        heads.append(jnp.dot(p.astype(jnp.bfloat16), vh,
                             preferred_element_type=F32))
    o = jnp.concatenate(heads, axis=1)
    proj = _bdot_t(o, ow_ref[...]) + ob_ref[...]
    x = x_ref[...]
    if scale is not None:
        x = x * scale + p_ref[...]
    o_ref[...] = _ln(x + proj, lw_ref[...], lb_ref[...])


def _attn_ln(x, pe, qkv, ow, ob, lw, lb, scale, nhead, last_only=False,
             sb=256):
    S, D = x.shape
    dh = D // nhead
    nb = S // sb
    last = nb - 1
    xmap = (lambda i: (last, 0)) if last_only else (lambda i: (i, 0))
    args = [x] + ([pe] if scale is not None else []) + [
        qkv, qkv, qkv, ow, ob.reshape(1, D), lw.reshape(1, D),
        lb.reshape(1, D)]
    pe_spec = ([pl.BlockSpec((sb, D), xmap)] if scale is not None else [])
    kw = dict(sb=sb, S=S, nhead=nhead, dh=dh, scale=scale,
              last_only=last_only)
    if scale is not None:
        body = functools.partial(_attnln_body, **kw)
    else:
        def body(x_ref, q_ref, k_ref, v_ref, ow_ref, ob_ref, lw_ref,
                 lb_ref, o_ref):
            _attnln_body(x_ref, None, q_ref, k_ref, v_ref, ow_ref, ob_ref,
                         lw_ref, lb_ref, o_ref, **kw)
    return pl.pallas_call(
        body,
        grid=(1 if last_only else nb,),
        in_specs=[pl.BlockSpec((sb, D), xmap)] + pe_spec + [
            pl.BlockSpec((sb, D), xmap),
            pl.BlockSpec((S, D), lambda i: (0, 1)),
            pl.BlockSpec((S, D), lambda i: (0, 2)),
            pl.BlockSpec((D, D), lambda i: (0, 0)),
            pl.BlockSpec((1, D), lambda i: (0, 0)),
            pl.BlockSpec((1, D), lambda i: (0, 0)),
            pl.BlockSpec((1, D), lambda i: (0, 0)),
        ],
        out_specs=pl.BlockSpec((sb, D), (lambda i: (0, 0)) if last_only
                               else (lambda i: (i, 0))),
        out_shape=jax.ShapeDtypeStruct((sb if last_only else S, D), F32),
    )(*args)


def _ff(x, w1_ref, b1_ref, w2_ref, b2_ref):
    h = jnp.maximum(_bdot_t(x, w1_ref[...]) + b1_ref[...], 0.0)
    return _bdot_t(h, w2_ref[...]) + b2_ref[...]


def _ffln_body(x_ref, w1_ref, b1_ref, w2_ref, b2_ref, lw_ref, lb_ref, o_ref):
    x = x_ref[...]
    f = _ff(x, w1_ref, b1_ref, w2_ref, b2_ref)
    o_ref[...] = _ln(x + f, lw_ref[...], lb_ref[...])


def _ffln_specs(D, H, NG=None):
    sp = [
        pl.BlockSpec((H, D), lambda i: (0, 0)),
        pl.BlockSpec((1, H), lambda i: (0, 0)),
        pl.BlockSpec((D, H), lambda i: (0, 0)),
        pl.BlockSpec((1, D), lambda i: (0, 0)),
        pl.BlockSpec((1, D), lambda i: (0, 0)),
        pl.BlockSpec((1, D), lambda i: (0, 0)),
    ]
    if NG is not None:
        sp += [pl.BlockSpec((NG, D), lambda i: (0, 0)),
               pl.BlockSpec((1, NG), lambda i: (0, 0))]
    return sp


def _ff_ln(x, w1, b1, w2, b2, lw, lb, mb=256):
    M, D = x.shape
    H, _ = w1.shape
    return pl.pallas_call(
        _ffln_body,
        grid=(M // mb,),
        in_specs=[pl.BlockSpec((mb, D), lambda i: (i, 0))] + _ffln_specs(D, H),
        out_specs=pl.BlockSpec((mb, D), lambda i: (i, 0)),
        out_shape=jax.ShapeDtypeStruct((M, D), F32),
    )(x, w1, b1.reshape(1, H), w2, b2.reshape(1, D), lw.reshape(1, D),
      lb.reshape(1, D))


def _fflngate_body(x_ref, w1_ref, b1_ref, w2_ref, b2_ref, lw_ref, lb_ref,
                   gw_ref, gb_ref, s_ref, o_ref, *, nq, mb):
    x = x_ref[...]
    f = _ff(x, w1_ref, b1_ref, w2_ref, b2_ref)
    x2 = _ln(x + f, lw_ref[...], lb_ref[...])
    xl = x2[mb - 1:mb, :]
    sc = lax.dot_general(xl, gw_ref[...], (((1,), (1,)), ((), ())),
                         preferred_element_type=F32) + gb_ref[...]
    sc = 1.0 / (1.0 + jnp.exp(-sc))
    s_ref[...] = sc
    R, C = sc.shape
    flat = (lax.broadcasted_iota(jnp.int32, (R, C), 0) * C
            + lax.broadcasted_iota(jnp.int32, (R, C), 1))
    rowi = lax.broadcasted_iota(jnp.int32, (64, 128), 0)
    coli = lax.broadcasted_iota(jnp.int32, (64, 128), 1)

    def body(t, carry):
        s, o = carry
        m = jnp.max(s)
        cand = jnp.where(s == m, flat, jnp.int32(2 ** 30))
        amin = jnp.min(cand)
        o = o + jnp.where(rowi == t, m, 0.0)
        s = jnp.where(flat == amin, jnp.float32(-1e30), s)
        return s, o

    sc, o = lax.fori_loop(0, nq, body, (sc, jnp.zeros((64, 128), F32)))
    total = jnp.sum(jnp.where(coli == 0, o, 0.0))
    o_ref[...] = o / total


def _ff_ln_gate(xlast, w1, b1, w2, b2, lw, lb, gw, gb, nq):
    mb, D = xlast.shape
    H, _ = w1.shape
    NG, _ = gw.shape
    return pl.pallas_call(
        functools.partial(_fflngate_body, nq=nq, mb=mb),
        grid=(1,),
        in_specs=[pl.BlockSpec((mb, D), lambda i: (0, 0))]
        + _ffln_specs(D, H, NG),
        out_specs=(pl.BlockSpec((1, NG), lambda i: (0, 0)),
                   pl.BlockSpec((64, 128), lambda i: (0, 0))),
        out_shape=(jax.ShapeDtypeStruct((1, NG), F32),
                   jax.ShapeDtypeStruct((64, 128), F32)),
    )(xlast, w1, b1.reshape(1, H), w2, b2.reshape(1, D), lw.reshape(1, D),
      lb.reshape(1, D), gw, gb.reshape(1, NG))


def _contrib_body(w_ref, rv_ref, ri_ref, oc_ref, oi_ref, *, sb, npair, nrow):
    rv = rv_ref[...]
    ri = ri_ref[...]
    cps = []
    ips = []
    for j in range(npair):
        a = jnp.log(w_ref[2 * j, 0, 0] * rv[2 * j, 0] + 1e-40)
        b = jnp.log(w_ref[2 * j + 1, 0, 0] * rv[2 * j + 1, 0] + 1e-40)
        cps.append(jnp.concatenate([a, b], axis=1)[:, None, :])
        ips.append(jnp.concatenate([ri[2 * j, 0], ri[2 * j + 1, 0]],
                                   axis=1)[:, None, :])
    pad = nrow - npair
    cps.append(jnp.zeros((sb, pad, 128), F32))
    ips.append(jnp.zeros((sb, pad, 128), jnp.int32))
    oc_ref[...] = jnp.concatenate(cps, axis=1)
    oi_ref[...] = jnp.concatenate(ips, axis=1)


def _contrib(w2d, rv4, ri4, sb=128, nrow=32):
    NQ_, _, S_, TK = rv4.shape
    npair = NQ_ // 2
    w3 = w2d.reshape(64, 1, 128)
    return pl.pallas_call(
        functools.partial(_contrib_body, sb=sb, npair=npair, nrow=nrow),
        grid=(S_ // sb,),
        in_specs=[
            pl.BlockSpec((64, 1, 128), lambda s: (0, 0, 0)),
            pl.BlockSpec((NQ_, 1, sb, TK), lambda s: (0, 0, s, 0)),
            pl.BlockSpec((NQ_, 1, sb, TK), lambda s: (0, 0, s, 0)),
        ],
        out_specs=(pl.BlockSpec((sb, nrow, 128), lambda s: (s, 0, 0)),
                   pl.BlockSpec((sb, nrow, 128), lambda s: (s, 0, 0))),
        out_shape=(jax.ShapeDtypeStruct((S_, nrow, 128), F32),
                   jax.ShapeDtypeStruct((S_, nrow, 128), jnp.int32)),
    )(w3, rv4, ri4)


def _loss_body(p_ref, a_ref, o_ref, *, V):
    part = jnp.sum(p_ref[...], axis=1, keepdims=True)
    alab = a_ref[...][:, 0:1]
    lr = jnp.log(V * math.e + part) - 1.0 - alab
    o_ref[...] = jnp.mean(lr).reshape(1, 1)


def _loss(part, alab, V):
    return pl.pallas_call(
        functools.partial(_loss_body, V=V),
        out_shape=jax.ShapeDtypeStruct((1, 1), F32),
    )(part, alab)


def _sc_mesh():
    return plsc.VectorSubcoreMesh(
        core_axis_name="c", subcore_axis_name="s",
        num_cores=_NC, num_subcores=_NS)


def _sc_embed_gather(idx, emb):
    (Sn,) = idx.shape
    V, D = emb.shape
    bpw = Sn // _NW

    @functools.partial(
        pl.kernel, mesh=_sc_mesh(),
        out_type=jax.ShapeDtypeStruct((Sn, D), F32),
        scratch_types=[
            pltpu.VMEM((bpw,), jnp.int32),
            pltpu.VMEM((bpw, D), F32),
            pltpu.SemaphoreType.DMA,
        ],
    )
    def k(idx_hbm, emb_hbm, out_hbm, idx_v, rows_v, sem):
        wid = lax.axis_index("s") * _NC + lax.axis_index("c")
        base = wid * bpw
        pltpu.sync_copy(idx_hbm.at[pl.ds(base, bpw)], idx_v)
        pltpu.async_copy(emb_hbm.at[idx_v], rows_v, sem).wait()
        pltpu.sync_copy(rows_v, out_hbm.at[pl.ds(base, bpw)])

    return k(idx, emb)


def _sc_ce(carr, iarr, labels, V, NQ_):
    (Sn,) = labels.shape
    _, nrow, _ = carr.shape
    npair = NQ_ // 2
    bpw = Sn // _NW
    Vp = ((V + _NL - 1) // _NL) * _NL
    nzero = Vp // _NL
    E = math.e
    R = 2
    nblk = bpw // R
    nck = 128 // _NL

    @functools.partial(
        pl.kernel, mesh=_sc_mesh(),
        compiler_params=pltpu.CompilerParams(needs_layout_passes=False),
        out_type=(jax.ShapeDtypeStruct((Sn, _NL), F32),
                  jax.ShapeDtypeStruct((Sn, _NL), F32)),
        scratch_types=[
            pltpu.VMEM((Vp,), F32),
            pltpu.VMEM((R, nrow, 128), F32),
            pltpu.VMEM((R, nrow, 128), jnp.int32),
            pltpu.VMEM((R, nrow, 128), F32),
            pltpu.VMEM((R, nrow, 128), jnp.int32),
            pltpu.VMEM((bpw,), jnp.int32),
            pltpu.VMEM((bpw, _NL), F32),
            pltpu.VMEM((bpw, _NL), F32),
            pltpu.SemaphoreType.DMA,
            pltpu.SemaphoreType.DMA,
            pltpu.SemaphoreType.DMA,
            pltpu.SemaphoreType.DMA,
        ],
    )
    def k(c_hbm, i_hbm, lab_hbm, part_hbm, alab_hbm,
          accum, vbufa, ibufa, vbufb, ibufb, labv, pout, aout,
          sva, sia, svb, sib):
        wid = lax.axis_index("s") * _NC + lax.axis_index("c")
        base = wid * bpw
        pltpu.sync_copy(lab_hbm.at[pl.ds(base, bpw)], labv)
        zeros16 = jnp.zeros((_NL,), F32)
        lane16 = lax.iota(jnp.int32, _NL).astype(F32)

        def zbody(t, carry):
            accum[pl.ds(t * _NL, _NL)] = zeros16
            return carry

        lax.fori_loop(0, nzero, zbody, 0)

        def fire(t, vbuf, ibuf, sv, si):
            s0 = base + t * R
            pltpu.async_copy(c_hbm.at[pl.ds(s0, R)], vbuf, sv)
            pltpu.async_copy(i_hbm.at[pl.ds(s0, R)], ibuf, si)

        def process(t, vbuf, ibuf, sv, si):
            pltpu.make_async_copy(c_hbm.at[pl.ds(0, R)], vbuf, sv).wait()
            pltpu.make_async_copy(i_hbm.at[pl.ds(0, R)], ibuf, si).wait()
            for r in range(R):
                i = t * R + r

                @plsc.parallel_loop(0, npair, 1, carry=jnp.int32(0))
                def pass_a(j, carry):
                    for kk in range(nck):
                        iv = ibuf[r, j, pl.ds(kk * _NL, _NL)]
                        vv = vbuf[r, j, pl.ds(kk * _NL, _NL)]
                        plsc.addupdate_scatter(accum, [iv], vv)
                    return carry

                ivec = jnp.full((_NL,), i, jnp.int32)
                lab = plsc.load_gather(labv, [ivec])
                aout[i] = plsc.load_gather(accum, [lab])

                def pass_b(j, acc):
                    for kk in range(nck):
                        iv = ibuf[r, j, pl.ds(kk * _NL, _NL)]
                        a = plsc.load_gather(accum, [iv])
                        plsc.store_scatter(accum, [iv], lane16)
                        win = plsc.load_gather(accum, [iv]) == lane16
                        term = jnp.exp(a + 1.0) - E
                        acc = acc + jnp.where(win, term, 0.0)
                        plsc.store_scatter(accum, [iv], zeros16)
                    return acc

                acc = lax.fori_loop(0, npair, pass_b, jnp.zeros((_NL,), F32))
                pout[i] = acc

        fire(0, vbufa, ibufa, sva, sia)

        def blk2(u, carry):
            t0 = 2 * u
            fire(t0 + 1, vbufb, ibufb, svb, sib)
            process(t0, vbufa, ibufa, sva, sia)

            @pl.when(t0 + 2 < nblk)
            def _():
                fire(t0 + 2, vbufa, ibufa, sva, sia)

            process(t0 + 1, vbufb, ibufb, svb, sib)
            return carry

        lax.fori_loop(0, nblk // 2, blk2, 0)
        pltpu.sync_copy(pout, part_hbm.at[pl.ds(base, bpw)])
        pltpu.sync_copy(aout, alab_hbm.at[pl.ds(base, bpw)])

    return k(carr, iarr, labels)


def kernel(inputs, response_values, response_indices, emb, gates_w, gates_b, layers):
    B_, S_ = inputs.shape
    V_, D_ = emb.shape
    NQ_, _, _, TK = response_values.shape
    nhead = 2
    nhid = layers[0]["ff1_w"].shape[0]
    nhid_p = 256

    idx = inputs.reshape(S_).astype(jnp.int32)
    x = _sc_embed_gather(idx, emb)
    pe = _posenc(S_, D_)
    scale = math.sqrt(D_)

    nl = len(layers)
    for li, p in enumerate(layers):
        sc = scale if li == 0 else None
        last = li == nl - 1
        qkv = _qkv(x, pe, p["in_w"], p["in_b"], sc)
        x1 = _attn_ln(x, pe, qkv, p["out_w"], p["out_b"],
                      p["ln1_w"], p["ln1_b"], sc, nhead, last_only=last)
        f1w = jnp.zeros((nhid_p, D_), F32).at[:nhid].set(p["ff1_w"])
        f1b = jnp.zeros((nhid_p,), F32).at[:nhid].set(p["ff1_b"])
        f2w = jnp.zeros((D_, nhid_p), F32).at[:, :nhid].set(p["ff2_w"])
        if last:
            score, w2d = _ff_ln_gate(x1, f1w, f1b, f2w, p["ff2_b"],
                                     p["ln2_w"], p["ln2_b"],
                                     gates_w, gates_b, NQ_)
        else:
            x = _ff_ln(x1, f1w, f1b, f2w, p["ff2_b"], p["ln2_w"], p["ln2_b"])

    routing_score = score.reshape(-1)
    carr, iarr = _contrib(w2d, response_values,
                          response_indices.astype(jnp.int32))
    part, alab = _sc_ce(carr, iarr, idx, V_, NQ_)
    loss = _loss(part, alab, V_)
    return loss.reshape(()), routing_score


# speedup vs baseline: 1.4032x; 1.0263x over previous
import functools
import math

import numpy as np
import jax
import jax.numpy as jnp
from jax import lax
from jax.experimental import pallas as pl
from jax.experimental.pallas import tpu as pltpu
from jax.experimental.pallas import tpu_sc as plsc

F32 = jnp.float32

_NC, _NS, _NL = 2, 16, 16
_NW = _NC * _NS


def _posenc(seq, dim):
    pos = np.arange(seq)[:, None].astype(np.float32)
    div = np.exp(np.arange(0, dim, 2).astype(np.float32) * (-math.log(10000.0) / dim))
    pe = np.zeros((seq, dim), np.float32)
    pe[:, 0::2] = np.sin(pos * div)
    pe[:, 1::2] = np.cos(pos * div)
    return jnp.asarray(pe)


def _ln(x, w, b):
    m = jnp.mean(x, axis=1, keepdims=True)
    var = jnp.mean((x - m) ** 2, axis=1, keepdims=True)
    return (x - m) / jnp.sqrt(var + 1e-5) * w + b


def _bdot_t(a, b):
    return lax.dot_general(a.astype(jnp.bfloat16), b.astype(jnp.bfloat16),
                           (((1,), (1,)), ((), ())),
                           preferred_element_type=F32)


def _qkv_body(x_ref, p_ref, w_ref, b_ref, o_ref, *, scale):
    x = x_ref[...]
    if scale is not None:
        x = x * scale + p_ref[...]
    o_ref[...] = _bdot_t(x, w_ref[...]) + b_ref[...]


def _qkv(x, pe, w, b, scale, mb=256):
    M, D = x.shape
    N, _ = w.shape
    args = [x] + ([pe] if scale is not None else []) + [w, b.reshape(1, N)]
    pe_spec = ([pl.BlockSpec((mb, D), lambda i: (i, 0))]
               if scale is not None else [])
    body = (functools.partial(_qkv_body, scale=scale) if scale is not None
            else (lambda x_ref, w_ref, b_ref, o_ref:
                  _qkv_body(x_ref, None, w_ref, b_ref, o_ref, scale=None)))
    return pl.pallas_call(
        body,
        grid=(M // mb,),
        in_specs=[pl.BlockSpec((mb, D), lambda i: (i, 0))] + pe_spec + [
            pl.BlockSpec((N, D), lambda i: (0, 0)),
            pl.BlockSpec((1, N), lambda i: (0, 0)),
        ],
        out_specs=pl.BlockSpec((mb, N), lambda i: (i, 0)),
        out_shape=jax.ShapeDtypeStruct((M, N), F32),
    )(*args)


def _attn_part(x, pe_ref, q_ref, k_ref, v_ref, ow_ref, ob_ref,
               l1w_ref, l1b_ref, *, row0, sb, S, nhead, dh, scale):
    rows = lax.broadcasted_iota(jnp.int32, (sb, S), 0) + row0
    cols = lax.broadcasted_iota(jnp.int32, (sb, S), 1)
    neg = jnp.float32(-1e30)
    rs = 1.0 / math.sqrt(dh)
    q = q_ref[...]
    heads = []
    for h in range(nhead):
        qh = q[:, h * dh:(h + 1) * dh].astype(jnp.bfloat16)
        kh = k_ref[...][:, h * dh:(h + 1) * dh].astype(jnp.bfloat16)
        vh = v_ref[...][:, h * dh:(h + 1) * dh].astype(jnp.bfloat16)
        s = lax.dot_general(qh, kh, (((1,), (1,)), ((), ())),
                            preferred_element_type=F32) * rs
        s = jnp.where(cols > rows, neg, s)
        m = jnp.max(s, axis=1, keepdims=True)
        p = jnp.exp(s - m)
        p = p / jnp.sum(p, axis=1, keepdims=True)
        heads.append(jnp.dot(p.astype(jnp.bfloat16), vh,
                             preferred_element_type=F32))
    o = jnp.concatenate(heads, axis=1)
    proj = _bdot_t(o, ow_ref[...]) + ob_ref[...]
    if scale is not None:
        x = x * scale + pe_ref[...]
    return _ln(x + proj, l1w_ref[...], l1b_ref[...])


def _ff(x, w1_ref, b1_ref, w2_ref, b2_ref):
    h = jnp.maximum(_bdot_t(x, w1_ref[...]) + b1_ref[...], 0.0)
    return _bdot_t(h, w2_ref[...]) + b2_ref[...]


def _layer_specs(sb, S, D, H, xmap, scale):
    pe_spec = ([pl.BlockSpec((sb, D), xmap)] if scale is not None else [])
    return [pl.BlockSpec((sb, D), xmap)] + pe_spec + [
        pl.BlockSpec((sb, D), xmap),
        pl.BlockSpec((S, D), lambda i: (0, 1)),
        pl.BlockSpec((S, D), lambda i: (0, 2)),
        pl.BlockSpec((D, D), lambda i: (0, 0)),
        pl.BlockSpec((1, D), lambda i: (0, 0)),
        pl.BlockSpec((1, D), lambda i: (0, 0)),
        pl.BlockSpec((1, D), lambda i: (0, 0)),
        pl.BlockSpec((H, D), lambda i: (0, 0)),
        pl.BlockSpec((1, H), lambda i: (0, 0)),
        pl.BlockSpec((D, H), lambda i: (0, 0)),
        pl.BlockSpec((1, D), lambda i: (0, 0)),
        pl.BlockSpec((1, D), lambda i: (0, 0)),
        pl.BlockSpec((1, D), lambda i: (0, 0)),
    ]


def _layer_args(x, pe, qkv, p, w1, b1, w2, D, H, scale):
    return [x] + ([pe] if scale is not None else []) + [
        qkv, qkv, qkv, p["out_w"], p["out_b"].reshape(1, D),
        p["ln1_w"].reshape(1, D), p["ln1_b"].reshape(1, D),
        w1, b1.reshape(1, H), w2, p["ff2_b"].reshape(1, D),
        p["ln2_w"].reshape(1, D), p["ln2_b"].reshape(1, D)]


def _layer_qkv_body(x_ref, p_ref, q_ref, k_ref, v_ref, ow_ref, ob_ref,
                    l1w_ref, l1b_ref, w1_ref, b1_ref, w2_ref, b2_ref,
                    l2w_ref, l2b_ref, nw_ref, nb_ref, x2_ref, qkv2_ref,
                    *, sb, S, nhead, dh, scale):
    row0 = pl.program_id(0) * sb
    x1 = _attn_part(x_ref[...], p_ref, q_ref, k_ref, v_ref, ow_ref, ob_ref,
                    l1w_ref, l1b_ref, row0=row0, sb=sb, S=S, nhead=nhead,
                    dh=dh, scale=scale)
    x2 = _ln(x1 + _ff(x1, w1_ref, b1_ref, w2_ref, b2_ref),
             l2w_ref[...], l2b_ref[...])
    x2_ref[...] = x2
    qkv2_ref[...] = _bdot_t(x2, nw_ref[...]) + nb_ref[...]


def _layer_qkv(x, pe, qkv, p, w1, b1, w2, nw, nb2, scale, nhead, sb=256):
    S, D = x.shape
    dh = D // nhead
    H, _ = w1.shape
    N3, _ = nw.shape
    xmap = lambda i: (i, 0)

    def body(*refs):
        if scale is None:
            refs = refs[:1] + (None,) + refs[1:]
        _layer_qkv_body(*refs, sb=sb, S=S, nhead=nhead, dh=dh, scale=scale)

    specs = _layer_specs(sb, S, D, H, xmap, scale) + [
        pl.BlockSpec((N3, D), lambda i: (0, 0)),
        pl.BlockSpec((1, N3), lambda i: (0, 0)),
    ]
    return pl.pallas_call(
        body,
        grid=(S // sb,),
        in_specs=specs,
        out_specs=(pl.BlockSpec((sb, D), xmap),
                   pl.BlockSpec((sb, N3), xmap)),
        out_shape=(jax.ShapeDtypeStruct((S, D), F32),
                   jax.ShapeDtypeStruct((S, N3), F32)),
    )(*(_layer_args(x, pe, qkv, p, w1, b1, w2, D, H, scale)
        + [nw, nb2.reshape(1, N3)]))


def _layer_gate_body(x_ref, p_ref, q_ref, k_ref, v_ref, ow_ref, ob_ref,
                     l1w_ref, l1b_ref, w1_ref, b1_ref, w2_ref, b2_ref,
                     l2w_ref, l2b_ref, gw_ref, gb_ref, s_ref, o_ref,
                     *, sb, S, nhead, dh, scale, nq):
    x1 = _attn_part(x_ref[...], p_ref, q_ref, k_ref, v_ref, ow_ref, ob_ref,
                    l1w_ref, l1b_ref, row0=S - sb, sb=sb, S=S, nhead=nhead,
                    dh=dh, scale=scale)
    x2 = _ln(x1 + _ff(x1, w1_ref, b1_ref, w2_ref, b2_ref),
             l2w_ref[...], l2b_ref[...])
    xl = x2[sb - 1:sb, :]
    sc = lax.dot_general(xl, gw_ref[...], (((1,), (1,)), ((), ())),
                         preferred_element_type=F32) + gb_ref[...]
    sc = 1.0 / (1.0 + jnp.exp(-sc))
    s_ref[...] = sc
    R, C = sc.shape
    flat = (lax.broadcasted_iota(jnp.int32, (R, C), 0) * C
            + lax.broadcasted_iota(jnp.int32, (R, C), 1))
    rowi = lax.broadcasted_iota(jnp.int32, (64, 128), 0)
    coli = lax.broadcasted_iota(jnp.int32, (64, 128), 1)

    def body(t, carry):
        s, o = carry
        m = jnp.max(s)
        cand = jnp.where(s == m, flat, jnp.int32(2 ** 30))
        amin = jnp.min(cand)
        o = o + jnp.where(rowi == t, m, 0.0)
        s = jnp.where(flat == amin, jnp.float32(-1e30), s)
        return s, o

    sc, o = lax.fori_loop(0, nq, body, (sc, jnp.zeros((64, 128), F32)))
    total = jnp.sum(jnp.where(coli == 0, o, 0.0))
    o_ref[...] = o / total


def _layer_gate(x, pe, qkv, p, w1, b1, w2, gw, gb, scale, nhead, nq,
                sb=256):
    S, D = x.shape
    dh = D // nhead
    H, _ = w1.shape
    NG, _ = gw.shape
    last = S // sb - 1
    xmap = lambda i: (last, 0)

    def body(*refs):
        if scale is None:
            refs = refs[:1] + (None,) + refs[1:]
        _layer_gate_body(*refs, sb=sb, S=S, nhead=nhead, dh=dh,
                         scale=scale, nq=nq)

    specs = _layer_specs(sb, S, D, H, xmap, scale) + [
        pl.BlockSpec((NG, D), lambda i: (0, 0)),
        pl.BlockSpec((1, NG), lambda i: (0, 0)),
    ]
    return pl.pallas_call(
        body,
        grid=(1,),
        in_specs=specs,
        out_specs=(pl.BlockSpec((1, NG), lambda i: (0, 0)),
                   pl.BlockSpec((64, 128), lambda i: (0, 0))),
        out_shape=(jax.ShapeDtypeStruct((1, NG), F32),
                   jax.ShapeDtypeStruct((64, 128), F32)),
    )(*(_layer_args(x, pe, qkv, p, w1, b1, w2, D, H, scale) + [gw, gb.reshape(1, NG)]))


def _contrib_body(w_ref, rv_ref, ri_ref, oc_ref, oi_ref, *, sb, npair, nrow):
    rv = rv_ref[...]
    ri = ri_ref[...]
    cps = []
    ips = []
    for j in range(npair):
        a = jnp.log(w_ref[2 * j, 0, 0] * rv[2 * j, 0] + 1e-40)
        b = jnp.log(w_ref[2 * j + 1, 0, 0] * rv[2 * j + 1, 0] + 1e-40)
        cps.append(jnp.concatenate([a, b], axis=1)[:, None, :])
        ips.append(jnp.concatenate([ri[2 * j, 0], ri[2 * j + 1, 0]],
                                   axis=1)[:, None, :])
    pad = nrow - npair
    cps.append(jnp.zeros((sb, pad, 128), F32))
    ips.append(jnp.zeros((sb, pad, 128), jnp.int32))
    oc_ref[...] = jnp.concatenate(cps, axis=1)
    oi_ref[...] = jnp.concatenate(ips, axis=1)


def _contrib(w2d, rv4, ri4, sb=128, nrow=32):
    NQ_, _, S_, TK = rv4.shape
    npair = NQ_ // 2
    w3 = w2d.reshape(64, 1, 128)
    return pl.pallas_call(
        functools.partial(_contrib_body, sb=sb, npair=npair, nrow=nrow),
        grid=(S_ // sb,),
        in_specs=[
            pl.BlockSpec((64, 1, 128), lambda s: (0, 0, 0)),
            pl.BlockSpec((NQ_, 1, sb, TK), lambda s: (0, 0, s, 0)),
            pl.BlockSpec((NQ_, 1, sb, TK), lambda s: (0, 0, s, 0)),
        ],
        out_specs=(pl.BlockSpec((sb, nrow, 128), lambda s: (s, 0, 0)),
                   pl.BlockSpec((sb, nrow, 128), lambda s: (s, 0, 0))),
        out_shape=(jax.ShapeDtypeStruct((S_, nrow, 128), F32),
                   jax.ShapeDtypeStruct((S_, nrow, 128), jnp.int32)),
    )(w3, rv4, ri4)


def _loss_body(p_ref, a_ref, o_ref, *, V):
    part = jnp.sum(p_ref[...], axis=1, keepdims=True)
    alab = a_ref[...][:, 0:1]
    lr = jnp.log(V * math.e + part) - 1.0 - alab
    o_ref[...] = jnp.mean(lr).reshape(1, 1)


def _loss(part, alab, V):
    return pl.pallas_call(
        functools.partial(_loss_body, V=V),
        out_shape=jax.ShapeDtypeStruct((1, 1), F32),
    )(part, alab)


def _sc_mesh():
    return plsc.VectorSubcoreMesh(
        core_axis_name="c", subcore_axis_name="s",
        num_cores=_NC, num_subcores=_NS)


def _sc_embed_gather(idx, emb):
    (Sn,) = idx.shape
    V, D = emb.shape
    bpw = Sn // _NW

    @functools.partial(
        pl.kernel, mesh=_sc_mesh(),
        out_type=jax.ShapeDtypeStruct((Sn, D), F32),
        scratch_types=[
            pltpu.VMEM((bpw,), jnp.int32),
            pltpu.VMEM((bpw, D), F32),
            pltpu.SemaphoreType.DMA,
        ],
    )
    def k(idx_hbm, emb_hbm, out_hbm, idx_v, rows_v, sem):
        wid = lax.axis_index("s") * _NC + lax.axis_index("c")
        base = wid * bpw
        pltpu.sync_copy(idx_hbm.at[pl.ds(base, bpw)], idx_v)
        pltpu.async_copy(emb_hbm.at[idx_v], rows_v, sem).wait()
        pltpu.sync_copy(rows_v, out_hbm.at[pl.ds(base, bpw)])

    return k(idx, emb)


def _sc_ce(carr, iarr, labels, V, NQ_):
    (Sn,) = labels.shape
    _, nrow, _ = carr.shape
    npair = NQ_ // 2
    bpw = Sn // _NW
    Vp = ((V + _NL - 1) // _NL) * _NL
    nzero = Vp // _NL
    E = math.e
    R = 2
    nblk = bpw // R
    nck = 128 // _NL

    @functools.partial(
        pl.kernel, mesh=_sc_mesh(),
        compiler_params=pltpu.CompilerParams(needs_layout_passes=False),
        out_type=(jax.ShapeDtypeStruct((Sn, _NL), F32),
                  jax.ShapeDtypeStruct((Sn, _NL), F32)),
        scratch_types=[
            pltpu.VMEM((Vp,), F32),
            pltpu.VMEM((R, nrow, 128), F32),
            pltpu.VMEM((R, nrow, 128), jnp.int32),
            pltpu.VMEM((R, nrow, 128), F32),
            pltpu.VMEM((R, nrow, 128), jnp.int32),
            pltpu.VMEM((bpw,), jnp.int32),
            pltpu.VMEM((bpw, _NL), F32),
            pltpu.VMEM((bpw, _NL), F32),
            pltpu.SemaphoreType.DMA,
            pltpu.SemaphoreType.DMA,
            pltpu.SemaphoreType.DMA,
            pltpu.SemaphoreType.DMA,
        ],
    )
    def k(c_hbm, i_hbm, lab_hbm, part_hbm, alab_hbm,
          accum, vbufa, ibufa, vbufb, ibufb, labv, pout, aout,
          sva, sia, svb, sib):
        wid = lax.axis_index("s") * _NC + lax.axis_index("c")
        base = wid * bpw
        pltpu.sync_copy(lab_hbm.at[pl.ds(base, bpw)], labv)
        zeros16 = jnp.zeros((_NL,), F32)
        lane16 = lax.iota(jnp.int32, _NL).astype(F32)

        def zbody(t, carry):
            accum[pl.ds(t * _NL, _NL)] = zeros16
            return carry

        lax.fori_loop(0, nzero, zbody, 0)

        def fire(t, vbuf, ibuf, sv, si):
            s0 = base + t * R
            pltpu.async_copy(c_hbm.at[pl.ds(s0, R)], vbuf, sv)
            pltpu.async_copy(i_hbm.at[pl.ds(s0, R)], ibuf, si)

        def process(t, vbuf, ibuf, sv, si):
            pltpu.make_async_copy(c_hbm.at[pl.ds(0, R)], vbuf, sv).wait()
            pltpu.make_async_copy(i_hbm.at[pl.ds(0, R)], ibuf, si).wait()
            for r in range(R):
                i = t * R + r

                @plsc.parallel_loop(0, npair, 1, carry=jnp.int32(0))
                def pass_a(j, carry):
                    for kk in range(nck):
                        iv = ibuf[r, j, pl.ds(kk * _NL, _NL)]
                        vv = vbuf[r, j, pl.ds(kk * _NL, _NL)]
                        plsc.addupdate_scatter(accum, [iv], vv)
                    return carry

                ivec = jnp.full((_NL,), i, jnp.int32)
                lab = plsc.load_gather(labv, [ivec])
                aout[i] = plsc.load_gather(accum, [lab])

                def pass_b(j, acc):
                    for kk in range(nck):
                        iv = ibuf[r, j, pl.ds(kk * _NL, _NL)]
                        a = plsc.load_gather(accum, [iv])
                        plsc.store_scatter(accum, [iv], lane16)
                        win = plsc.load_gather(accum, [iv]) == lane16
                        term = jnp.exp(a + 1.0) - E
                        acc = acc + jnp.where(win, term, 0.0)
                        plsc.store_scatter(accum, [iv], zeros16)
                    return acc

                acc = lax.fori_loop(0, npair, pass_b, jnp.zeros((_NL,), F32))
                pout[i] = acc

        fire(0, vbufa, ibufa, sva, sia)

        def blk2(u, carry):
            t0 = 2 * u
            fire(t0 + 1, vbufb, ibufb, svb, sib)
            process(t0, vbufa, ibufa, sva, sia)

            @pl.when(t0 + 2 < nblk)
            def _():
                fire(t0 + 2, vbufa, ibufa, sva, sia)

            process(t0 + 1, vbufb, ibufb, svb, sib)
            return carry

        lax.fori_loop(0, nblk // 2, blk2, 0)
        pltpu.sync_copy(pout, part_hbm.at[pl.ds(base, bpw)])
        pltpu.sync_copy(aout, alab_hbm.at[pl.ds(base, bpw)])

    return k(carr, iarr, labels)


def kernel(inputs, response_values, response_indices, emb, gates_w, gates_b, layers):
    B_, S_ = inputs.shape
    V_, D_ = emb.shape
    NQ_, _, _, TK = response_values.shape
    nhead = 2
    nhid = layers[0]["ff1_w"].shape[0]
    nhid_p = 256

    idx = inputs.reshape(S_).astype(jnp.int32)
    x = _sc_embed_gather(idx, emb)
    pe = _posenc(S_, D_)
    scale = math.sqrt(D_)

    nl = len(layers)
    qkv = _qkv(x, pe, layers[0]["in_w"], layers[0]["in_b"], scale)
    for li, p in enumerate(layers):
        sc = scale if li == 0 else None
        f1w = jnp.zeros((nhid_p, D_), F32).at[:nhid].set(p["ff1_w"])
        f1b = jnp.zeros((nhid_p,), F32).at[:nhid].set(p["ff1_b"])
        f2w = jnp.zeros((D_, nhid_p), F32).at[:, :nhid].set(p["ff2_w"])
        if li == nl - 1:
            score, w2d = _layer_gate(x, pe, qkv, p, f1w, f1b, f2w,
                                     gates_w, gates_b, sc, nhead, NQ_)
        else:
            nxt = layers[li + 1]
            x, qkv = _layer_qkv(x, pe, qkv, p, f1w, f1b, f2w,
                                nxt["in_w"], nxt["in_b"], sc, nhead)

    routing_score = score.reshape(-1)
    carr, iarr = _contrib(w2d, response_values,
                          response_indices.astype(jnp.int32))
    part, alab = _sc_ce(carr, iarr, idx, V_, NQ_)
    loss = _loss(part, alab, V_)
    return loss.reshape(()), routing_score


# speedup vs baseline: 1.4454x; 1.0300x over previous
import functools
import math

import numpy as np
import jax
import jax.numpy as jnp
from jax import lax
from jax.experimental import pallas as pl
from jax.experimental.pallas import tpu as pltpu
from jax.experimental.pallas import tpu_sc as plsc

F32 = jnp.float32

_NC, _NS, _NL = 2, 16, 16
_NW = _NC * _NS


def _posenc(seq, dim):
    pos = np.arange(seq)[:, None].astype(np.float32)
    div = np.exp(np.arange(0, dim, 2).astype(np.float32) * (-math.log(10000.0) / dim))
    pe = np.zeros((seq, dim), np.float32)
    pe[:, 0::2] = np.sin(pos * div)
    pe[:, 1::2] = np.cos(pos * div)
    return jnp.asarray(pe)


def _ln(x, w, b):
    m = jnp.mean(x, axis=1, keepdims=True)
    var = jnp.mean((x - m) ** 2, axis=1, keepdims=True)
    return (x - m) / jnp.sqrt(var + 1e-5) * w + b


def _bdot_t(a, b):
    return lax.dot_general(a.astype(jnp.bfloat16), b.astype(jnp.bfloat16),
                           (((1,), (1,)), ((), ())),
                           preferred_element_type=F32)


def _qkv_body(x_ref, p_ref, w_ref, b_ref, o_ref, *, scale):
    x = x_ref[...]
    if scale is not None:
        x = x * scale + p_ref[...]
    o_ref[...] = _bdot_t(x, w_ref[...]) + b_ref[...]


def _qkv(x, pe, w, b, scale, mb=256):
    M, D = x.shape
    N, _ = w.shape
    args = [x] + ([pe] if scale is not None else []) + [w, b.reshape(1, N)]
    pe_spec = ([pl.BlockSpec((mb, D), lambda i: (i, 0))]
               if scale is not None else [])
    body = (functools.partial(_qkv_body, scale=scale) if scale is not None
            else (lambda x_ref, w_ref, b_ref, o_ref:
                  _qkv_body(x_ref, None, w_ref, b_ref, o_ref, scale=None)))
    return pl.pallas_call(
        body,
        grid=(M // mb,),
        in_specs=[pl.BlockSpec((mb, D), lambda i: (i, 0))] + pe_spec + [
            pl.BlockSpec((N, D), lambda i: (0, 0)),
            pl.BlockSpec((1, N), lambda i: (0, 0)),
        ],
        out_specs=pl.BlockSpec((mb, N), lambda i: (i, 0)),
        out_shape=jax.ShapeDtypeStruct((M, N), F32),
    )(*args)


def _attn_part(x, pe_ref, q_ref, k_ref, v_ref, ow_ref, ob_ref,
               l1w_ref, l1b_ref, *, row0, sb, S, nhead, dh, scale):
    rows = lax.broadcasted_iota(jnp.int32, (sb, S), 0) + row0
    cols = lax.broadcasted_iota(jnp.int32, (sb, S), 1)
    neg = jnp.float32(-1e30)
    rs = 1.0 / math.sqrt(dh)
    q = q_ref[...]
    heads = []
    for h in range(nhead):
        qh = q[:, h * dh:(h + 1) * dh].astype(jnp.bfloat16)
        kh = k_ref[...][:, h * dh:(h + 1) * dh].astype(jnp.bfloat16)
        vh = v_ref[...][:, h * dh:(h + 1) * dh].astype(jnp.bfloat16)
        s = lax.dot_general(qh, kh, (((1,), (1,)), ((), ())),
                            preferred_element_type=F32) * rs
        s = jnp.where(cols > rows, neg, s)
        m = jnp.max(s, axis=1, keepdims=True)
        p = jnp.exp(s - m)
        p = p / jnp.sum(p, axis=1, keepdims=True)
        heads.append(jnp.dot(p.astype(jnp.bfloat16), vh,
                             preferred_element_type=F32))
    o = jnp.concatenate(heads, axis=1)
    proj = _bdot_t(o, ow_ref[...]) + ob_ref[...]
    if scale is not None:
        x = x * scale + pe_ref[...]
    return _ln(x + proj, l1w_ref[...], l1b_ref[...])


def _ff(x, w1_ref, b1_ref, w2_ref, b2_ref):
    h = jnp.maximum(_bdot_t(x, w1_ref[...]) + b1_ref[...], 0.0)
    return _bdot_t(h, w2_ref[...]) + b2_ref[...]


def _layer_specs(sb, S, D, H, xmap, scale):
    pe_spec = ([pl.BlockSpec((sb, D), xmap)] if scale is not None else [])
    return [pl.BlockSpec((sb, D), xmap)] + pe_spec + [
        pl.BlockSpec((sb, D), xmap),
        pl.BlockSpec((S, D), lambda i: (0, 1)),
        pl.BlockSpec((S, D), lambda i: (0, 2)),
        pl.BlockSpec((D, D), lambda i: (0, 0)),
        pl.BlockSpec((1, D), lambda i: (0, 0)),
        pl.BlockSpec((1, D), lambda i: (0, 0)),
        pl.BlockSpec((1, D), lambda i: (0, 0)),
        pl.BlockSpec((H, D), lambda i: (0, 0)),
        pl.BlockSpec((1, H), lambda i: (0, 0)),
        pl.BlockSpec((D, H), lambda i: (0, 0)),
        pl.BlockSpec((1, D), lambda i: (0, 0)),
        pl.BlockSpec((1, D), lambda i: (0, 0)),
        pl.BlockSpec((1, D), lambda i: (0, 0)),
    ]


def _layer_args(x, pe, qkv, p, w1, b1, w2, D, H, scale):
    return [x] + ([pe] if scale is not None else []) + [
        qkv, qkv, qkv, p["out_w"], p["out_b"].reshape(1, D),
        p["ln1_w"].reshape(1, D), p["ln1_b"].reshape(1, D),
        w1, b1.reshape(1, H), w2, p["ff2_b"].reshape(1, D),
        p["ln2_w"].reshape(1, D), p["ln2_b"].reshape(1, D)]


def _layer_qkv_body(x_ref, p_ref, q_ref, k_ref, v_ref, ow_ref, ob_ref,
                    l1w_ref, l1b_ref, w1_ref, b1_ref, w2_ref, b2_ref,
                    l2w_ref, l2b_ref, nw_ref, nb_ref, x2_ref, qkv2_ref,
                    *, sb, S, nhead, dh, scale):
    row0 = pl.program_id(0) * sb
    x1 = _attn_part(x_ref[...], p_ref, q_ref, k_ref, v_ref, ow_ref, ob_ref,
                    l1w_ref, l1b_ref, row0=row0, sb=sb, S=S, nhead=nhead,
                    dh=dh, scale=scale)
    x2 = _ln(x1 + _ff(x1, w1_ref, b1_ref, w2_ref, b2_ref),
             l2w_ref[...], l2b_ref[...])
    x2_ref[...] = x2
    qkv2_ref[...] = _bdot_t(x2, nw_ref[...]) + nb_ref[...]


def _layer_qkv(x, pe, qkv, p, w1, b1, w2, nw, nb2, scale, nhead, sb=256):
    S, D = x.shape
    dh = D // nhead
    H, _ = w1.shape
    N3, _ = nw.shape
    xmap = lambda i: (i, 0)

    def body(*refs):
        if scale is None:
            refs = refs[:1] + (None,) + refs[1:]
        _layer_qkv_body(*refs, sb=sb, S=S, nhead=nhead, dh=dh, scale=scale)

    specs = _layer_specs(sb, S, D, H, xmap, scale) + [
        pl.BlockSpec((N3, D), lambda i: (0, 0)),
        pl.BlockSpec((1, N3), lambda i: (0, 0)),
    ]
    return pl.pallas_call(
        body,
        grid=(S // sb,),
        in_specs=specs,
        out_specs=(pl.BlockSpec((sb, D), xmap),
                   pl.BlockSpec((sb, N3), xmap)),
        out_shape=(jax.ShapeDtypeStruct((S, D), F32),
                   jax.ShapeDtypeStruct((S, N3), F32)),
    )(*(_layer_args(x, pe, qkv, p, w1, b1, w2, D, H, scale)
        + [nw, nb2.reshape(1, N3)]))


def _layer_gate_body(x_ref, p_ref, q_ref, k_ref, v_ref, ow_ref, ob_ref,
                     l1w_ref, l1b_ref, w1_ref, b1_ref, w2_ref, b2_ref,
                     l2w_ref, l2b_ref, gw_ref, gb_ref, s_ref, o_ref,
                     *, sb, S, nhead, dh, scale, nq):
    x1 = _attn_part(x_ref[...], p_ref, q_ref, k_ref, v_ref, ow_ref, ob_ref,
                    l1w_ref, l1b_ref, row0=S - sb, sb=sb, S=S, nhead=nhead,
                    dh=dh, scale=scale)
    x2 = _ln(x1 + _ff(x1, w1_ref, b1_ref, w2_ref, b2_ref),
             l2w_ref[...], l2b_ref[...])
    xl = x2[sb - 1:sb, :]
    sc = lax.dot_general(xl, gw_ref[...], (((1,), (1,)), ((), ())),
                         preferred_element_type=F32) + gb_ref[...]
    sc = 1.0 / (1.0 + jnp.exp(-sc))
    s_ref[...] = sc
    R, C = sc.shape
    flat = (lax.broadcasted_iota(jnp.int32, (R, C), 0) * C
            + lax.broadcasted_iota(jnp.int32, (R, C), 1))
    rowi = lax.broadcasted_iota(jnp.int32, (64, 128), 0)
    coli = lax.broadcasted_iota(jnp.int32, (64, 128), 1)

    def body(t, carry):
        s, o = carry
        m = jnp.max(s)
        cand = jnp.where(s == m, flat, jnp.int32(2 ** 30))
        amin = jnp.min(cand)
        o = o + jnp.where(rowi == t, m, 0.0)
        s = jnp.where(flat == amin, jnp.float32(-1e30), s)
        return s, o

    sc, o = lax.fori_loop(0, nq, body, (sc, jnp.zeros((64, 128), F32)))
    total = jnp.sum(jnp.where(coli == 0, o, 0.0))
    o_ref[...] = o / total


def _layer_gate(x, pe, qkv, p, w1, b1, w2, gw, gb, scale, nhead, nq,
                sb=256):
    S, D = x.shape
    dh = D // nhead
    H, _ = w1.shape
    NG, _ = gw.shape
    last = S // sb - 1
    xmap = lambda i: (last, 0)

    def body(*refs):
        if scale is None:
            refs = refs[:1] + (None,) + refs[1:]
        _layer_gate_body(*refs, sb=sb, S=S, nhead=nhead, dh=dh,
                         scale=scale, nq=nq)

    specs = _layer_specs(sb, S, D, H, xmap, scale) + [
        pl.BlockSpec((NG, D), lambda i: (0, 0)),
        pl.BlockSpec((1, NG), lambda i: (0, 0)),
    ]
    return pl.pallas_call(
        body,
        grid=(1,),
        in_specs=specs,
        out_specs=(pl.BlockSpec((1, NG), lambda i: (0, 0)),
                   pl.BlockSpec((64, 128), lambda i: (0, 0))),
        out_shape=(jax.ShapeDtypeStruct((1, NG), F32),
                   jax.ShapeDtypeStruct((64, 128), F32)),
    )(*(_layer_args(x, pe, qkv, p, w1, b1, w2, D, H, scale) + [gw, gb.reshape(1, NG)]))


def _contrib_body(w_ref, rv_ref, ri_ref, oc_ref, oi_ref, *, sb, npair, nrow):
    rv = rv_ref[...]
    ri = ri_ref[...]
    cps = []
    ips = []
    for j in range(npair):
        a = jnp.log(w_ref[2 * j, 0, 0] * rv[2 * j, 0] + 1e-40)
        b = jnp.log(w_ref[2 * j + 1, 0, 0] * rv[2 * j + 1, 0] + 1e-40)
        cps.append(jnp.concatenate([a, b], axis=1)[:, None, :])
        ips.append(jnp.concatenate([ri[2 * j, 0], ri[2 * j + 1, 0]],
                                   axis=1)[:, None, :])
    pad = nrow - npair
    cps.append(jnp.zeros((sb, pad, 128), F32))
    ips.append(jnp.zeros((sb, pad, 128), jnp.int32))
    oc_ref[...] = jnp.concatenate(cps, axis=1)
    oi_ref[...] = jnp.concatenate(ips, axis=1)


def _contrib(w2d, rv4, ri4, sb=128, nrow=32):
    NQ_, _, S_, TK = rv4.shape
    npair = NQ_ // 2
    w3 = w2d.reshape(64, 1, 128)
    return pl.pallas_call(
        functools.partial(_contrib_body, sb=sb, npair=npair, nrow=nrow),
        grid=(S_ // sb,),
        in_specs=[
            pl.BlockSpec((64, 1, 128), lambda s: (0, 0, 0)),
            pl.BlockSpec((NQ_, 1, sb, TK), lambda s: (0, 0, s, 0)),
            pl.BlockSpec((NQ_, 1, sb, TK), lambda s: (0, 0, s, 0)),
        ],
        out_specs=(pl.BlockSpec((sb, nrow, 128), lambda s: (s, 0, 0)),
                   pl.BlockSpec((sb, nrow, 128), lambda s: (s, 0, 0))),
        out_shape=(jax.ShapeDtypeStruct((S_, nrow, 128), F32),
                   jax.ShapeDtypeStruct((S_, nrow, 128), jnp.int32)),
    )(w3, rv4, ri4)


def _loss_body(p_ref, a_ref, o_ref, *, V):
    part = jnp.sum(p_ref[...], axis=1, keepdims=True)
    alab = a_ref[...][:, 0:1]
    lr = jnp.log(V * math.e + part) - 1.0 - alab
    o_ref[...] = jnp.mean(lr).reshape(1, 1)


def _loss(part, alab, V):
    return pl.pallas_call(
        functools.partial(_loss_body, V=V),
        out_shape=jax.ShapeDtypeStruct((1, 1), F32),
    )(part, alab)


def _sc_mesh():
    return plsc.VectorSubcoreMesh(
        core_axis_name="c", subcore_axis_name="s",
        num_cores=_NC, num_subcores=_NS)


def _sc_embed_gather(idx, emb):
    (Sn,) = idx.shape
    V, D = emb.shape
    bpw = Sn // _NW

    @functools.partial(
        pl.kernel, mesh=_sc_mesh(),
        out_type=jax.ShapeDtypeStruct((Sn, D), F32),
        scratch_types=[
            pltpu.VMEM((bpw,), jnp.int32),
            pltpu.VMEM((bpw, D), F32),
            pltpu.SemaphoreType.DMA,
        ],
    )
    def k(idx_hbm, emb_hbm, out_hbm, idx_v, rows_v, sem):
        wid = lax.axis_index("s") * _NC + lax.axis_index("c")
        base = wid * bpw
        pltpu.sync_copy(idx_hbm.at[pl.ds(base, bpw)], idx_v)
        pltpu.async_copy(emb_hbm.at[idx_v], rows_v, sem).wait()
        pltpu.sync_copy(rows_v, out_hbm.at[pl.ds(base, bpw)])

    return k(idx, emb)


def _sc_ce(carr, iarr, labels, V, NQ_):
    (Sn,) = labels.shape
    _, nrow, _ = carr.shape
    npair = NQ_ // 2
    bpw = Sn // _NW
    Vp = ((V + _NL - 1) // _NL) * _NL
    nzero = Vp // _NL
    E = math.e
    R = 2
    nblk = bpw // R
    nck = 128 // _NL

    @functools.partial(
        pl.kernel, mesh=_sc_mesh(),
        compiler_params=pltpu.CompilerParams(needs_layout_passes=False),
        out_type=(jax.ShapeDtypeStruct((Sn, _NL), F32),
                  jax.ShapeDtypeStruct((Sn, _NL), F32)),
        scratch_types=[
            pltpu.VMEM((Vp,), F32),
            pltpu.VMEM((R, nrow, 128), F32),
            pltpu.VMEM((R, nrow, 128), jnp.int32),
            pltpu.VMEM((R, nrow, 128), F32),
            pltpu.VMEM((R, nrow, 128), jnp.int32),
            pltpu.VMEM((bpw,), jnp.int32),
            pltpu.VMEM((bpw, _NL), F32),
            pltpu.VMEM((bpw, _NL), F32),
            pltpu.SemaphoreType.DMA,
            pltpu.SemaphoreType.DMA,
            pltpu.SemaphoreType.DMA,
            pltpu.SemaphoreType.DMA,
        ],
    )
    def k(c_hbm, i_hbm, lab_hbm, part_hbm, alab_hbm,
          accum, vbufa, ibufa, vbufb, ibufb, labv, pout, aout,
          sva, sia, svb, sib):
        wid = lax.axis_index("s") * _NC + lax.axis_index("c")
        base = wid * bpw
        pltpu.sync_copy(lab_hbm.at[pl.ds(base, bpw)], labv)
        zeros16 = jnp.zeros((_NL,), F32)
        lane16 = lax.iota(jnp.int32, _NL).astype(F32)

        def zbody(t, carry):
            accum[pl.ds(t * _NL, _NL)] = zeros16
            return carry

        lax.fori_loop(0, nzero, zbody, 0)

        def fire(t, vbuf, ibuf, sv, si):
            s0 = base + t * R
            pltpu.async_copy(c_hbm.at[pl.ds(s0, R)], vbuf, sv)
            pltpu.async_copy(i_hbm.at[pl.ds(s0, R)], ibuf, si)

        def process(t, vbuf, ibuf, sv, si):
            pltpu.make_async_copy(c_hbm.at[pl.ds(0, R)], vbuf, sv).wait()
            pltpu.make_async_copy(i_hbm.at[pl.ds(0, R)], ibuf, si).wait()
            for r in range(R):
                i = t * R + r

                @plsc.parallel_loop(0, npair, 1, carry=jnp.int32(0))
                def pass_a(j, carry):
                    for kk in range(nck):
                        iv = ibuf[r, j, pl.ds(kk * _NL, _NL)]
                        vv = vbuf[r, j, pl.ds(kk * _NL, _NL)]
                        plsc.addupdate_scatter(accum, [iv], vv)
                    return carry

                ivec = jnp.full((_NL,), i, jnp.int32)
                lab = plsc.load_gather(labv, [ivec])
                aout[i] = plsc.load_gather(accum, [lab])

                def pass_b(j, acc):
                    for kk in range(nck):
                        iv = ibuf[r, j, pl.ds(kk * _NL, _NL)]
                        a = plsc.load_gather(accum, [iv])
                        _, lastm = plsc.scan_count(iv)
                        term = jnp.exp(a + 1.0) - E
                        acc = acc + jnp.where(lastm, term, 0.0)
                        plsc.store_scatter(accum, [iv], zeros16)
                    return acc

                acc = lax.fori_loop(0, npair, pass_b, jnp.zeros((_NL,), F32))
                pout[i] = acc

        fire(0, vbufa, ibufa, sva, sia)

        def blk2(u, carry):
            t0 = 2 * u
            fire(t0 + 1, vbufb, ibufb, svb, sib)
            process(t0, vbufa, ibufa, sva, sia)

            @pl.when(t0 + 2 < nblk)
            def _():
                fire(t0 + 2, vbufa, ibufa, sva, sia)

            process(t0 + 1, vbufb, ibufb, svb, sib)
            return carry

        lax.fori_loop(0, nblk // 2, blk2, 0)
        pltpu.sync_copy(pout, part_hbm.at[pl.ds(base, bpw)])
        pltpu.sync_copy(aout, alab_hbm.at[pl.ds(base, bpw)])

    return k(carr, iarr, labels)


def kernel(inputs, response_values, response_indices, emb, gates_w, gates_b, layers):
    B_, S_ = inputs.shape
    V_, D_ = emb.shape
    NQ_, _, _, TK = response_values.shape
    nhead = 2
    nhid = layers[0]["ff1_w"].shape[0]
    nhid_p = 256

    idx = inputs.reshape(S_).astype(jnp.int32)
    x = _sc_embed_gather(idx, emb)
    pe = _posenc(S_, D_)
    scale = math.sqrt(D_)

    nl = len(layers)
    qkv = _qkv(x, pe, layers[0]["in_w"], layers[0]["in_b"], scale)
    for li, p in enumerate(layers):
        sc = scale if li == 0 else None
        f1w = jnp.zeros((nhid_p, D_), F32).at[:nhid].set(p["ff1_w"])
        f1b = jnp.zeros((nhid_p,), F32).at[:nhid].set(p["ff1_b"])
        f2w = jnp.zeros((D_, nhid_p), F32).at[:, :nhid].set(p["ff2_w"])
        if li == nl - 1:
            score, w2d = _layer_gate(x, pe, qkv, p, f1w, f1b, f2w,
                                     gates_w, gates_b, sc, nhead, NQ_)
        else:
            nxt = layers[li + 1]
            x, qkv = _layer_qkv(x, pe, qkv, p, f1w, f1b, f2w,
                                nxt["in_w"], nxt["in_b"], sc, nhead)

    routing_score = score.reshape(-1)
    carr, iarr = _contrib(w2d, response_values,
                          response_indices.astype(jnp.int32))
    part, alab = _sc_ce(carr, iarr, idx, V_, NQ_)
    loss = _loss(part, alab, V_)
    return loss.reshape(()), routing_score


# speedup vs baseline: 1.4459x; 1.0003x over previous
import functools
import math

import numpy as np
import jax
import jax.numpy as jnp
from jax import lax
from jax.experimental import pallas as pl
from jax.experimental.pallas import tpu as pltpu
from jax.experimental.pallas import tpu_sc as plsc

F32 = jnp.float32

_NC, _NS, _NL = 2, 16, 16
_NW = _NC * _NS


def _posenc(seq, dim):
    pos = np.arange(seq)[:, None].astype(np.float32)
    div = np.exp(np.arange(0, dim, 2).astype(np.float32) * (-math.log(10000.0) / dim))
    pe = np.zeros((seq, dim), np.float32)
    pe[:, 0::2] = np.sin(pos * div)
    pe[:, 1::2] = np.cos(pos * div)
    return jnp.asarray(pe)


def _ln(x, w, b):
    m = jnp.mean(x, axis=1, keepdims=True)
    var = jnp.mean((x - m) ** 2, axis=1, keepdims=True)
    return (x - m) / jnp.sqrt(var + 1e-5) * w + b


def _bdot_t(a, b):
    return lax.dot_general(a.astype(jnp.bfloat16), b.astype(jnp.bfloat16),
                           (((1,), (1,)), ((), ())),
                           preferred_element_type=F32)


def _qkv_body(x_ref, p_ref, w_ref, b_ref, o_ref, *, scale):
    x = x_ref[...]
    if scale is not None:
        x = x * scale + p_ref[...]
    o_ref[...] = _bdot_t(x, w_ref[...]) + b_ref[...]


def _qkv(x, pe, w, b, scale, mb=256):
    M, D = x.shape
    N, _ = w.shape
    args = [x] + ([pe] if scale is not None else []) + [w, b.reshape(1, N)]
    pe_spec = ([pl.BlockSpec((mb, D), lambda i: (i, 0))]
               if scale is not None else [])
    body = (functools.partial(_qkv_body, scale=scale) if scale is not None
            else (lambda x_ref, w_ref, b_ref, o_ref:
                  _qkv_body(x_ref, None, w_ref, b_ref, o_ref, scale=None)))
    return pl.pallas_call(
        body,
        grid=(M // mb,),
        in_specs=[pl.BlockSpec((mb, D), lambda i: (i, 0))] + pe_spec + [
            pl.BlockSpec((N, D), lambda i: (0, 0)),
            pl.BlockSpec((1, N), lambda i: (0, 0)),
        ],
        out_specs=pl.BlockSpec((mb, N), lambda i: (i, 0)),
        out_shape=jax.ShapeDtypeStruct((M, N), F32),
    )(*args)


def _attn_part(x, pe_ref, q_ref, k_ref, v_ref, ow_ref, ob_ref,
               l1w_ref, l1b_ref, *, row0, sb, S, nhead, dh, scale):
    rows = lax.broadcasted_iota(jnp.int32, (sb, S), 0) + row0
    cols = lax.broadcasted_iota(jnp.int32, (sb, S), 1)
    neg = jnp.float32(-1e30)
    rs = 1.0 / math.sqrt(dh)
    q = q_ref[...]
    heads = []
    for h in range(nhead):
        qh = q[:, h * dh:(h + 1) * dh].astype(jnp.bfloat16)
        kh = k_ref[...][:, h * dh:(h + 1) * dh].astype(jnp.bfloat16)
        vh = v_ref[...][:, h * dh:(h + 1) * dh].astype(jnp.bfloat16)
        s = lax.dot_general(qh, kh, (((1,), (1,)), ((), ())),
                            preferred_element_type=F32) * rs
        s = jnp.where(cols > rows, neg, s)
        m = jnp.max(s, axis=1, keepdims=True)
        p = jnp.exp(s - m)
        p = p / jnp.sum(p, axis=1, keepdims=True)
        heads.append(jnp.dot(p.astype(jnp.bfloat16), vh,
                             preferred_element_type=F32))
    o = jnp.concatenate(heads, axis=1)
    proj = _bdot_t(o, ow_ref[...]) + ob_ref[...]
    if scale is not None:
        x = x * scale + pe_ref[...]
    return _ln(x + proj, l1w_ref[...], l1b_ref[...])


def _ff(x, w1_ref, b1_ref, w2_ref, b2_ref):
    h = jnp.maximum(_bdot_t(x, w1_ref[...]) + b1_ref[...], 0.0)
    return _bdot_t(h, w2_ref[...]) + b2_ref[...]


def _layer_specs(sb, S, D, H, xmap, scale):
    pe_spec = ([pl.BlockSpec((sb, D), xmap)] if scale is not None else [])
    return [pl.BlockSpec((sb, D), xmap)] + pe_spec + [
        pl.BlockSpec((sb, D), xmap),
        pl.BlockSpec((S, D), lambda i: (0, 1)),
        pl.BlockSpec((S, D), lambda i: (0, 2)),
        pl.BlockSpec((D, D), lambda i: (0, 0)),
        pl.BlockSpec((1, D), lambda i: (0, 0)),
        pl.BlockSpec((1, D), lambda i: (0, 0)),
        pl.BlockSpec((1, D), lambda i: (0, 0)),
        pl.BlockSpec((H, D), lambda i: (0, 0)),
        pl.BlockSpec((1, H), lambda i: (0, 0)),
        pl.BlockSpec((D, H), lambda i: (0, 0)),
        pl.BlockSpec((1, D), lambda i: (0, 0)),
        pl.BlockSpec((1, D), lambda i: (0, 0)),
        pl.BlockSpec((1, D), lambda i: (0, 0)),
    ]


def _layer_args(x, pe, qkv, p, w1, b1, w2, D, H, scale):
    return [x] + ([pe] if scale is not None else []) + [
        qkv, qkv, qkv, p["out_w"], p["out_b"].reshape(1, D),
        p["ln1_w"].reshape(1, D), p["ln1_b"].reshape(1, D),
        w1, b1.reshape(1, H), w2, p["ff2_b"].reshape(1, D),
        p["ln2_w"].reshape(1, D), p["ln2_b"].reshape(1, D)]


def _layer_qkv_body(x_ref, p_ref, q_ref, k_ref, v_ref, ow_ref, ob_ref,
                    l1w_ref, l1b_ref, w1_ref, b1_ref, w2_ref, b2_ref,
                    l2w_ref, l2b_ref, nw_ref, nb_ref, x2_ref, qkv2_ref,
                    *, sb, S, nhead, dh, scale):
    row0 = pl.program_id(0) * sb
    x1 = _attn_part(x_ref[...], p_ref, q_ref, k_ref, v_ref, ow_ref, ob_ref,
                    l1w_ref, l1b_ref, row0=row0, sb=sb, S=S, nhead=nhead,
                    dh=dh, scale=scale)
    x2 = _ln(x1 + _ff(x1, w1_ref, b1_ref, w2_ref, b2_ref),
             l2w_ref[...], l2b_ref[...])
    x2_ref[...] = x2
    qkv2_ref[...] = _bdot_t(x2, nw_ref[...]) + nb_ref[...]


def _layer_qkv(x, pe, qkv, p, w1, b1, w2, nw, nb2, scale, nhead, sb=256):
    S, D = x.shape
    dh = D // nhead
    H, _ = w1.shape
    N3, _ = nw.shape
    xmap = lambda i: (i, 0)

    def body(*refs):
        if scale is None:
            refs = refs[:1] + (None,) + refs[1:]
        _layer_qkv_body(*refs, sb=sb, S=S, nhead=nhead, dh=dh, scale=scale)

    specs = _layer_specs(sb, S, D, H, xmap, scale) + [
        pl.BlockSpec((N3, D), lambda i: (0, 0)),
        pl.BlockSpec((1, N3), lambda i: (0, 0)),
    ]
    return pl.pallas_call(
        body,
        grid=(S // sb,),
        in_specs=specs,
        out_specs=(pl.BlockSpec((sb, D), xmap),
                   pl.BlockSpec((sb, N3), xmap)),
        out_shape=(jax.ShapeDtypeStruct((S, D), F32),
                   jax.ShapeDtypeStruct((S, N3), F32)),
    )(*(_layer_args(x, pe, qkv, p, w1, b1, w2, D, H, scale)
        + [nw, nb2.reshape(1, N3)]))


def _layer_gate_body(x_ref, p_ref, q_ref, k_ref, v_ref, ow_ref, ob_ref,
                     l1w_ref, l1b_ref, w1_ref, b1_ref, w2_ref, b2_ref,
                     l2w_ref, l2b_ref, gw_ref, gb_ref, s_ref, o_ref,
                     *, sb, S, nhead, dh, scale, nq):
    x1 = _attn_part(x_ref[...], p_ref, q_ref, k_ref, v_ref, ow_ref, ob_ref,
                    l1w_ref, l1b_ref, row0=S - sb, sb=sb, S=S, nhead=nhead,
                    dh=dh, scale=scale)
    x2 = _ln(x1 + _ff(x1, w1_ref, b1_ref, w2_ref, b2_ref),
             l2w_ref[...], l2b_ref[...])
    xl = x2[sb - 1:sb, :]
    sc = lax.dot_general(xl, gw_ref[...], (((1,), (1,)), ((), ())),
                         preferred_element_type=F32) + gb_ref[...]
    sc = 1.0 / (1.0 + jnp.exp(-sc))
    s_ref[...] = sc
    R, C = sc.shape
    flat = (lax.broadcasted_iota(jnp.int32, (R, C), 0) * C
            + lax.broadcasted_iota(jnp.int32, (R, C), 1))
    rowi = lax.broadcasted_iota(jnp.int32, (64, 128), 0)
    coli = lax.broadcasted_iota(jnp.int32, (64, 128), 1)

    def body(t, carry):
        s, o = carry
        m = jnp.max(s)
        cand = jnp.where(s == m, flat, jnp.int32(2 ** 30))
        amin = jnp.min(cand)
        o = o + jnp.where(rowi == t, m, 0.0)
        s = jnp.where(flat == amin, jnp.float32(-1e30), s)
        return s, o

    sc, o = lax.fori_loop(0, nq, body, (sc, jnp.zeros((64, 128), F32)))
    total = jnp.sum(jnp.where(coli == 0, o, 0.0))
    o_ref[...] = o / total


def _layer_gate(x, pe, qkv, p, w1, b1, w2, gw, gb, scale, nhead, nq,
                sb=256):
    S, D = x.shape
    dh = D // nhead
    H, _ = w1.shape
    NG, _ = gw.shape
    last = S // sb - 1
    xmap = lambda i: (last, 0)

    def body(*refs):
        if scale is None:
            refs = refs[:1] + (None,) + refs[1:]
        _layer_gate_body(*refs, sb=sb, S=S, nhead=nhead, dh=dh,
                         scale=scale, nq=nq)

    specs = _layer_specs(sb, S, D, H, xmap, scale) + [
        pl.BlockSpec((NG, D), lambda i: (0, 0)),
        pl.BlockSpec((1, NG), lambda i: (0, 0)),
    ]
    return pl.pallas_call(
        body,
        grid=(1,),
        in_specs=specs,
        out_specs=(pl.BlockSpec((1, NG), lambda i: (0, 0)),
                   pl.BlockSpec((64, 128), lambda i: (0, 0))),
        out_shape=(jax.ShapeDtypeStruct((1, NG), F32),
                   jax.ShapeDtypeStruct((64, 128), F32)),
    )(*(_layer_args(x, pe, qkv, p, w1, b1, w2, D, H, scale) + [gw, gb.reshape(1, NG)]))


def _contrib_body(w_ref, rv_ref, ri_ref, oc_ref, oi_ref, *, sb, npair, nrow):
    rv = rv_ref[...]
    ri = ri_ref[...]
    cps = []
    ips = []
    for j in range(npair):
        a = jnp.log(w_ref[2 * j, 0, 0] * rv[2 * j, 0] + 1e-40)
        b = jnp.log(w_ref[2 * j + 1, 0, 0] * rv[2 * j + 1, 0] + 1e-40)
        cps.append(jnp.concatenate([a, b], axis=1)[:, None, :])
        ips.append(jnp.concatenate([ri[2 * j, 0], ri[2 * j + 1, 0]],
                                   axis=1)[:, None, :])
    pad = nrow - npair
    cps.append(jnp.zeros((sb, pad, 128), F32))
    ips.append(jnp.zeros((sb, pad, 128), jnp.int32))
    oc_ref[...] = jnp.concatenate(cps, axis=1)
    oi_ref[...] = jnp.concatenate(ips, axis=1)


def _contrib(w2d, rv4, ri4, sb=128, nrow=32):
    NQ_, _, S_, TK = rv4.shape
    npair = NQ_ // 2
    w3 = w2d.reshape(64, 1, 128)
    return pl.pallas_call(
        functools.partial(_contrib_body, sb=sb, npair=npair, nrow=nrow),
        grid=(S_ // sb,),
        in_specs=[
            pl.BlockSpec((64, 1, 128), lambda s: (0, 0, 0)),
            pl.BlockSpec((NQ_, 1, sb, TK), lambda s: (0, 0, s, 0)),
            pl.BlockSpec((NQ_, 1, sb, TK), lambda s: (0, 0, s, 0)),
        ],
        out_specs=(pl.BlockSpec((sb, nrow, 128), lambda s: (s, 0, 0)),
                   pl.BlockSpec((sb, nrow, 128), lambda s: (s, 0, 0))),
        out_shape=(jax.ShapeDtypeStruct((S_, nrow, 128), F32),
                   jax.ShapeDtypeStruct((S_, nrow, 128), jnp.int32)),
    )(w3, rv4, ri4)


def _loss_body(p_ref, a_ref, o_ref, *, V):
    part = jnp.sum(p_ref[...], axis=1, keepdims=True)
    alab = a_ref[...][:, 0:1]
    lr = jnp.log(V * math.e + part) - 1.0 - alab
    o_ref[...] = jnp.mean(lr).reshape(1, 1)


def _loss(part, alab, V):
    return pl.pallas_call(
        functools.partial(_loss_body, V=V),
        out_shape=jax.ShapeDtypeStruct((1, 1), F32),
    )(part, alab)


def _sc_mesh():
    return plsc.VectorSubcoreMesh(
        core_axis_name="c", subcore_axis_name="s",
        num_cores=_NC, num_subcores=_NS)


def _sc_embed_gather(idx, emb):
    (Sn,) = idx.shape
    V, D = emb.shape
    bpw = Sn // _NW

    @functools.partial(
        pl.kernel, mesh=_sc_mesh(),
        out_type=jax.ShapeDtypeStruct((Sn, D), F32),
        scratch_types=[
            pltpu.VMEM((bpw,), jnp.int32),
            pltpu.VMEM((bpw, D), F32),
            pltpu.SemaphoreType.DMA,
        ],
    )
    def k(idx_hbm, emb_hbm, out_hbm, idx_v, rows_v, sem):
        wid = lax.axis_index("s") * _NC + lax.axis_index("c")
        base = wid * bpw
        pltpu.sync_copy(idx_hbm.at[pl.ds(base, bpw)], idx_v)
        pltpu.async_copy(emb_hbm.at[idx_v], rows_v, sem).wait()
        pltpu.sync_copy(rows_v, out_hbm.at[pl.ds(base, bpw)])

    return k(idx, emb)


def _sc_ce(carr, iarr, labels, V, NQ_):
    (Sn,) = labels.shape
    _, nrow, _ = carr.shape
    npair = NQ_ // 2
    bpw = Sn // _NW
    Vp = ((V + _NL - 1) // _NL) * _NL
    nzero = Vp // _NL
    E = math.e
    R = 2
    nblk = bpw // R
    nck = 128 // _NL

    @functools.partial(
        pl.kernel, mesh=_sc_mesh(),
        compiler_params=pltpu.CompilerParams(needs_layout_passes=False),
        out_type=(jax.ShapeDtypeStruct((Sn, _NL), F32),
                  jax.ShapeDtypeStruct((Sn, _NL), F32)),
        scratch_types=[
            pltpu.VMEM((Vp,), F32),
            pltpu.VMEM((R, nrow, 128), F32),
            pltpu.VMEM((R, nrow, 128), jnp.int32),
            pltpu.VMEM((R, nrow, 128), F32),
            pltpu.VMEM((R, nrow, 128), jnp.int32),
            pltpu.VMEM((bpw,), jnp.int32),
            pltpu.VMEM((bpw, _NL), F32),
            pltpu.VMEM((bpw, _NL), F32),
            pltpu.SemaphoreType.DMA,
            pltpu.SemaphoreType.DMA,
            pltpu.SemaphoreType.DMA,
            pltpu.SemaphoreType.DMA,
        ],
    )
    def k(c_hbm, i_hbm, lab_hbm, part_hbm, alab_hbm,
          accum, vbufa, ibufa, vbufb, ibufb, labv, pout, aout,
          sva, sia, svb, sib):
        wid = lax.axis_index("s") * _NC + lax.axis_index("c")
        base = wid * bpw
        pltpu.sync_copy(lab_hbm.at[pl.ds(base, bpw)], labv)
        zeros16 = jnp.zeros((_NL,), F32)

        def zbody(t, carry):
            accum[pl.ds(t * _NL, _NL)] = zeros16
            return carry

        lax.fori_loop(0, nzero, zbody, 0)

        def fire(t, vbuf, ibuf, sv, si):
            s0 = base + t * R
            pltpu.async_copy(c_hbm.at[pl.ds(s0, R)], vbuf, sv)
            pltpu.async_copy(i_hbm.at[pl.ds(s0, R)], ibuf, si)

        def process(t, vbuf, ibuf, sv, si):
            pltpu.make_async_copy(c_hbm.at[pl.ds(0, R)], vbuf, sv).wait()
            pltpu.make_async_copy(i_hbm.at[pl.ds(0, R)], ibuf, si).wait()
            for r in range(R):
                i = t * R + r

                @plsc.parallel_loop(0, npair, 1, unroll=5, carry=jnp.int32(0))
                def pass_a(j, carry):
                    for kk in range(nck):
                        iv = ibuf[r, j, pl.ds(kk * _NL, _NL)]
                        vv = vbuf[r, j, pl.ds(kk * _NL, _NL)]
                        plsc.addupdate_scatter(accum, [iv], vv)
                    return carry

                ivec = jnp.full((_NL,), i, jnp.int32)
                lab = plsc.load_gather(labv, [ivec])
                aout[i] = plsc.load_gather(accum, [lab])

                def pass_b(j, acc):
                    for kk in range(nck):
                        iv = ibuf[r, j, pl.ds(kk * _NL, _NL)]
                        a = plsc.load_gather(accum, [iv])
                        _, lastm = plsc.scan_count(iv)
                        term = jnp.exp(a + 1.0) - E
                        acc = acc + jnp.where(lastm, term, 0.0)
                        plsc.store_scatter(accum, [iv], zeros16)
                    return acc

                acc = lax.fori_loop(0, npair, pass_b, jnp.zeros((_NL,), F32))
                pout[i] = acc

        fire(0, vbufa, ibufa, sva, sia)

        def blk2(u, carry):
            t0 = 2 * u
            fire(t0 + 1, vbufb, ibufb, svb, sib)
            process(t0, vbufa, ibufa, sva, sia)

            @pl.when(t0 + 2 < nblk)
            def _():
                fire(t0 + 2, vbufa, ibufa, sva, sia)

            process(t0 + 1, vbufb, ibufb, svb, sib)
            return carry

        lax.fori_loop(0, nblk // 2, blk2, 0)
        pltpu.sync_copy(pout, part_hbm.at[pl.ds(base, bpw)])
        pltpu.sync_copy(aout, alab_hbm.at[pl.ds(base, bpw)])

    return k(carr, iarr, labels)


def kernel(inputs, response_values, response_indices, emb, gates_w, gates_b, layers):
    B_, S_ = inputs.shape
    V_, D_ = emb.shape
    NQ_, _, _, TK = response_values.shape
    nhead = 2
    nhid = layers[0]["ff1_w"].shape[0]
    nhid_p = 256

    idx = inputs.reshape(S_).astype(jnp.int32)
    x = _sc_embed_gather(idx, emb)
    pe = _posenc(S_, D_)
    scale = math.sqrt(D_)

    nl = len(layers)
    qkv = _qkv(x, pe, layers[0]["in_w"], layers[0]["in_b"], scale)
    for li, p in enumerate(layers):
        sc = scale if li == 0 else None
        f1w = jnp.zeros((nhid_p, D_), F32).at[:nhid].set(p["ff1_w"])
        f1b = jnp.zeros((nhid_p,), F32).at[:nhid].set(p["ff1_b"])
        f2w = jnp.zeros((D_, nhid_p), F32).at[:, :nhid].set(p["ff2_w"])
        if li == nl - 1:
            score, w2d = _layer_gate(x, pe, qkv, p, f1w, f1b, f2w,
                                     gates_w, gates_b, sc, nhead, NQ_)
        else:
            nxt = layers[li + 1]
            x, qkv = _layer_qkv(x, pe, qkv, p, f1w, f1b, f2w,
                                nxt["in_w"], nxt["in_b"], sc, nhead)

    routing_score = score.reshape(-1)
    carr, iarr = _contrib(w2d, response_values,
                          response_indices.astype(jnp.int32))
    part, alab = _sc_ce(carr, iarr, idx, V_, NQ_)
    loss = _loss(part, alab, V_)
    return loss.reshape(()), routing_score
